# baseline (speedup 1.0000x reference)
import functools

import jax
import jax.numpy as jnp
from jax import lax
from jax.experimental import pallas as pl
from jax.experimental.pallas import tpu as pltpu
from jax.experimental.pallas import tpu_sc as plsc

N = 10000
E = 320000
DIN = 128
DH = 256
H = 4
G = 64

NC = 2
NS = 16
NW = NC * NS
EP = 327680
EPW = EP // NW
CK = 128
NCH = EPW // CK
CKA = 32
NCHA = EPW // CKA
NA = N + 8
NPS = 624
TBASE = NS * NPS
ZTAIL = NA - TBASE
RTAIL = N - TBASE

BN = 400
NB = N // BN


def _sc_mesh():
  return plsc.VectorSubcoreMesh(core_axis_name="c", subcore_axis_name="s",
                                num_cores=NC, num_subcores=NS)


def _wid():
  return lax.axis_index("c") * NS + lax.axis_index("s")


def _zero_acc(zeros_hh, acc, s):
  pltpu.sync_copy(zeros_hh, acc.at[pl.ds(s * NPS, NPS)])

  @pl.when(s == 0)
  def _():
    pltpu.sync_copy(zeros_hh.at[pl.ds(0, ZTAIL)], acc.at[pl.ds(TBASE, ZTAIL)])


def _readout(acc, out_h, c, s):
  pltpu.sync_copy(acc.at[pl.ds(s * NPS, NPS)],
                  out_h.at[pl.ds(c * N + s * NPS, NPS)])

  @pl.when(s == 0)
  def _():
    pltpu.sync_copy(acc.at[pl.ds(TBASE, RTAIL)],
                    out_h.at[pl.ds(c * N + TBASE, RTAIL)])


def _sc_count(dst, ones_h, zeros_h):
  def body(dst_h, ones_hh, zeros_hh, out_h, id0, id1, ones_v, smi0, smi1, acc):
    c = lax.axis_index("c")
    s = lax.axis_index("s")
    wid = c * NS + s
    ebase = wid * EPW
    _zero_acc(zeros_hh, acc, s)
    pltpu.sync_copy(ones_hh, ones_v)
    plsc.subcore_barrier()

    def idx_start(i, idb, smi):
      pltpu.async_copy(dst_h.at[pl.ds(ebase + i * CK, CK)], idb, smi)

    def idx_wait(i, idb, smi):
      pltpu.make_async_copy(dst_h.at[pl.ds(ebase + i * CK, CK)],
                            idb, smi).wait()

    idx_start(0, id0, smi0)
    idx_start(1, id1, smi1)
    npair = NCH // 2

    def pair(j, carry):
      i0 = 2 * j
      idx_wait(i0, id0, smi0)
      pltpu.sync_copy(ones_v, acc.at[id0], add=True)

      @pl.when(j < npair - 1)
      def _():
        idx_start(i0 + 2, id0, smi0)

      idx_wait(i0 + 1, id1, smi1)
      pltpu.sync_copy(ones_v, acc.at[id1], add=True)

      @pl.when(j < npair - 1)
      def _():
        idx_start(i0 + 3, id1, smi1)

      return carry

    lax.fori_loop(0, npair, pair, 0)
    plsc.subcore_barrier()
    _readout(acc, out_h, c, s)

  return pl.kernel(
      body,
      out_type=jax.ShapeDtypeStruct((NC * N, 16), jnp.float32),
      mesh=_sc_mesh(),
      compiler_params=pltpu.CompilerParams(use_tc_tiling_on_sc=False),
      scratch_types=[
          pltpu.VMEM((CK,), jnp.int32),
          pltpu.VMEM((CK,), jnp.int32),
          pltpu.VMEM((CK, 16), jnp.float32),
          pltpu.SemaphoreType.DMA,
          pltpu.SemaphoreType.DMA,
          pltpu.VMEM_SHARED((NA, 16), jnp.float32),
      ],
  )(dst, ones_h, zeros_h)


def _sc_seg_rows(table, src, dst, zeros_h):
  def body(tab_h, src_h, dst_h, zeros_hh, out_h,
           is0, id0, r0, is1, id1, r1, smi0, smi1, smg0, smg1, acc):
    c = lax.axis_index("c")
    s = lax.axis_index("s")
    wid = c * NS + s
    ebase = wid * EPW
    _zero_acc(zeros_hh, acc, s)
    plsc.subcore_barrier()

    def idx_start(i, isb, idb, smi):
      b = ebase + i * CK
      pltpu.async_copy(src_h.at[pl.ds(b, CK)], isb, smi)
      pltpu.async_copy(dst_h.at[pl.ds(b, CK)], idb, smi)

    def idx_wait(i, isb, idb, smi):
      b = ebase + i * CK
      pltpu.make_async_copy(src_h.at[pl.ds(b, CK)], isb, smi).wait()
      pltpu.make_async_copy(dst_h.at[pl.ds(b, CK)], idb, smi).wait()

    pltpu.sync_copy(src_h.at[pl.ds(ebase, CK)], is0)
    pltpu.sync_copy(dst_h.at[pl.ds(ebase, CK)], id0)
    pltpu.async_copy(tab_h.at[is0], r0, smg0)
    idx_start(1, is1, id1, smi1)

    npair = NCH // 2

    def pair(j, carry):
      i0 = 2 * j
      idx_wait(i0 + 1, is1, id1, smi1)
      pltpu.async_copy(tab_h.at[is1], r1, smg1)
      pltpu.make_async_copy(tab_h.at[is0], r0, smg0).wait()
      pltpu.sync_copy(r0, acc.at[id0], add=True)

      @pl.when(j < npair - 1)
      def _():
        idx_start(i0 + 2, is0, id0, smi0)

      @pl.when(j < npair - 1)
      def _():
        idx_wait(i0 + 2, is0, id0, smi0)
        pltpu.async_copy(tab_h.at[is0], r0, smg0)

      pltpu.make_async_copy(tab_h.at[is1], r1, smg1).wait()
      pltpu.sync_copy(r1, acc.at[id1], add=True)

      @pl.when(j < npair - 1)
      def _():
        idx_start(i0 + 3, is1, id1, smi1)

      return carry

    lax.fori_loop(0, npair, pair, 0)
    plsc.subcore_barrier()
    _readout(acc, out_h, c, s)

  return pl.kernel(
      body,
      out_type=jax.ShapeDtypeStruct((NC * N, DIN), jnp.float32),
      mesh=_sc_mesh(),
      compiler_params=pltpu.CompilerParams(use_tc_tiling_on_sc=False),
      scratch_types=[
          pltpu.VMEM((CK,), jnp.int32),
          pltpu.VMEM((CK,), jnp.int32),
          pltpu.VMEM((CK, DIN), jnp.float32),
          pltpu.VMEM((CK,), jnp.int32),
          pltpu.VMEM((CK,), jnp.int32),
          pltpu.VMEM((CK, DIN), jnp.float32),
          pltpu.SemaphoreType.DMA,
          pltpu.SemaphoreType.DMA,
          pltpu.SemaphoreType.DMA,
          pltpu.SemaphoreType.DMA,
          pltpu.VMEM_SHARED((NA, DIN), jnp.float32),
      ],
  )(table, src, dst, zeros_h)


def _sc_seg_rows_b(table, src, dst, zeros_h):
  def body(tab_h, src_h, dst_h, zeros_hh, out_h,
           is0, id0, r0, is1, id1, r1, smi0, smi1, smg0, smg1, acc):
    c = lax.axis_index("c")
    s = lax.axis_index("s")
    wid = c * NS + s
    ebase = wid * EPW
    _zero_acc(zeros_hh, acc, s)
    plsc.subcore_barrier()

    def idx_start(i, isb, idb, smi):
      b = ebase + i * CK
      pltpu.async_copy(src_h.at[pl.ds(b, CK)], isb, smi)
      pltpu.async_copy(dst_h.at[pl.ds(b, CK)], idb, smi)

    def idx_wait(i, isb, idb, smi):
      b = ebase + i * CK
      pltpu.make_async_copy(src_h.at[pl.ds(b, CK)], isb, smi).wait()
      pltpu.make_async_copy(dst_h.at[pl.ds(b, CK)], idb, smi).wait()

    pltpu.sync_copy(src_h.at[pl.ds(ebase, CK)], is0)
    pltpu.sync_copy(dst_h.at[pl.ds(ebase, CK)], id0)
    pltpu.async_copy(tab_h.at[is0], r0, smg0)
    idx_start(1, is1, id1, smi1)

    npair = NCH // 2

    def pair(j, carry):
      i0 = 2 * j
      idx_wait(i0 + 1, is1, id1, smi1)
      pltpu.async_copy(tab_h.at[is1], r1, smg1)
      pltpu.make_async_copy(tab_h.at[is0], r0, smg0).wait()
      pltpu.sync_copy(r0, acc.at[id0], add=True)

      @pl.when(j < npair - 1)
      def _():
        idx_start(i0 + 2, is0, id0, smi0)

      @pl.when(j < npair - 1)
      def _():
        idx_wait(i0 + 2, is0, id0, smi0)
        pltpu.async_copy(tab_h.at[is0], r0, smg0)

      pltpu.make_async_copy(tab_h.at[is1], r1, smg1).wait()
      pltpu.sync_copy(r1, acc.at[id1], add=True)

      @pl.when(j < npair - 1)
      def _():
        idx_start(i0 + 3, is1, id1, smi1)

      return carry

    lax.fori_loop(0, npair, pair, 0)
    plsc.subcore_barrier()
    _readout(acc, out_h, c, s)

  return pl.kernel(
      body,
      out_type=jax.ShapeDtypeStruct((NC * N, 2 * DIN), jnp.bfloat16),
      mesh=_sc_mesh(),
      compiler_params=pltpu.CompilerParams(use_tc_tiling_on_sc=False),
      scratch_types=[
          pltpu.VMEM((CK,), jnp.int32),
          pltpu.VMEM((CK,), jnp.int32),
          pltpu.VMEM((CK, 2 * DIN), jnp.bfloat16),
          pltpu.VMEM((CK,), jnp.int32),
          pltpu.VMEM((CK,), jnp.int32),
          pltpu.VMEM((CK, 2 * DIN), jnp.bfloat16),
          pltpu.SemaphoreType.DMA,
          pltpu.SemaphoreType.DMA,
          pltpu.SemaphoreType.DMA,
          pltpu.SemaphoreType.DMA,
          pltpu.VMEM_SHARED((NA, 2 * DIN), jnp.bfloat16),
      ],
  )(table, src, dst, zeros_h)


def _sc_gat_logits(als8, ald8, cvec, src, dst, zeros_h):
  def body(als_h, ald_h, c_h, src_h, dst_h, zeros_hh,
           ex_h, out_h, idx_s, idx_d, asv, adv, exb, cv, acc):
    c = lax.axis_index("c")
    s = lax.axis_index("s")
    wid = c * NS + s
    _zero_acc(zeros_hh, acc, s)
    pltpu.sync_copy(c_h, cv)
    plsc.subcore_barrier()

    def chunk(i, carry):
      base = wid * EPW + i * CK
      pltpu.sync_copy(src_h.at[pl.ds(base, CK)], idx_s)
      pltpu.sync_copy(dst_h.at[pl.ds(base, CK)], idx_d)
      pltpu.sync_copy(als_h.at[idx_s], asv)
      pltpu.sync_copy(ald_h.at[idx_d], adv)
      cvv = cv[...]

      def erow(j, carry2):
        t = asv[j, :] + adv[j, :]
        t = jnp.maximum(t, 0.2 * t) - cvv
        exb[j, :] = jnp.exp(t)
        return carry2

      lax.fori_loop(0, CK, erow, 0)
      pltpu.sync_copy(exb, ex_h.at[pl.ds(base, CK)])
      pltpu.sync_copy(exb, acc.at[idx_d], add=True)
      return carry

    lax.fori_loop(0, NCH, chunk, 0)
    plsc.subcore_barrier()
    _readout(acc, out_h, c, s)

  return pl.kernel(
      body,
      out_type=[
          jax.ShapeDtypeStruct((EP, 16), jnp.float32),
          jax.ShapeDtypeStruct((NC * N, 16), jnp.float32),
      ],
      mesh=_sc_mesh(),
      compiler_params=pltpu.CompilerParams(use_tc_tiling_on_sc=False),
      scratch_types=[
          pltpu.VMEM((CK,), jnp.int32),
          pltpu.VMEM((CK,), jnp.int32),
          pltpu.VMEM((CK, 16), jnp.float32),
          pltpu.VMEM((CK, 16), jnp.float32),
          pltpu.VMEM((CK, 16), jnp.float32),
          pltpu.VMEM((16,), jnp.float32),
          pltpu.VMEM_SHARED((NA, 16), jnp.float32),
      ],
  )(als8, ald8, cvec, src, dst, zeros_h)


def _sc_gat_agg(hh8, exw, r8, src, dst, zeros_h):
  def body(hh_h, ex_h, r_h, src_h, dst_h, zeros_hh, out_h,
           is0, id0, r0, e0, v0, is1, id1, r1, e1, v1,
           smi0, smi1, smg0, smg1, y, acc):
    c = lax.axis_index("c")
    s = lax.axis_index("s")
    wid = c * NS + s
    ebase = wid * EPW
    _zero_acc(zeros_hh, acc, s)
    plsc.subcore_barrier()

    def idx_start(i, isb, idb, smi):
      b = ebase + i * CKA
      pltpu.async_copy(src_h.at[pl.ds(b, CKA)], isb, smi)
      pltpu.async_copy(dst_h.at[pl.ds(b, CKA)], idb, smi)

    def idx_wait(i, isb, idb, smi):
      b = ebase + i * CKA
      pltpu.make_async_copy(src_h.at[pl.ds(b, CKA)], isb, smi).wait()
      pltpu.make_async_copy(dst_h.at[pl.ds(b, CKA)], idb, smi).wait()

    def fetch_start(i, isb, idb, rb, eb, vb, smg):
      b = ebase + i * CKA
      pltpu.async_copy(hh_h.at[isb], rb, smg)
      pltpu.async_copy(ex_h.at[pl.ds(b, CKA)], eb, smg)
      pltpu.async_copy(r_h.at[idb], vb, smg)

    def fetch_wait(i, isb, idb, rb, eb, vb, smg):
      b = ebase + i * CKA
      pltpu.make_async_copy(hh_h.at[isb], rb, smg).wait()
      pltpu.make_async_copy(ex_h.at[pl.ds(b, CKA)], eb, smg).wait()
      pltpu.make_async_copy(r_h.at[idb], vb, smg).wait()

    def compute_scatter(rb, eb, vb, idb):
      def eblk(p, carry2):
        js = [4 * p, 4 * p + 1, 4 * p + 2, 4 * p + 3]
        avs = [eb[j, :] * vb[j, :] for j in js]

        def splat(av, h):
          return jnp.full((16,), av[h], jnp.float32).astype(jnp.bfloat16)

        w = [[splat(av, h) for h in range(4)] for av in avs]
        for cb in range(16):
          for q in range(4):
            j = js[q]
            v = (w[q][0] * rb[j, pl.ds(cb * 16, 16)]
                 + w[q][1] * rb[j, pl.ds(256 + cb * 16, 16)]
                 + w[q][2] * rb[j, pl.ds(512 + cb * 16, 16)]
                 + w[q][3] * rb[j, pl.ds(768 + cb * 16, 16)])
            y[j, pl.ds(cb * 16, 16)] = v
        return carry2

      lax.fori_loop(0, CKA // 4, eblk, 0)
      pltpu.sync_copy(y, acc.at[idb], add=True)

    pltpu.sync_copy(src_h.at[pl.ds(ebase, CKA)], is0)
    pltpu.sync_copy(dst_h.at[pl.ds(ebase, CKA)], id0)
    fetch_start(0, is0, id0, r0, e0, v0, smg0)
    idx_start(1, is1, id1, smi1)

    npair = NCHA // 2

    def pair(j, carry):
      i0 = 2 * j
      idx_wait(i0 + 1, is1, id1, smi1)
      fetch_start(i0 + 1, is1, id1, r1, e1, v1, smg1)
      fetch_wait(i0, is0, id0, r0, e0, v0, smg0)
      compute_scatter(r0, e0, v0, id0)

      @pl.when(j < npair - 1)
      def _():
        idx_start(i0 + 2, is0, id0, smi0)
        idx_wait(i0 + 2, is0, id0, smi0)
        fetch_start(i0 + 2, is0, id0, r0, e0, v0, smg0)

      fetch_wait(i0 + 1, is1, id1, r1, e1, v1, smg1)
      compute_scatter(r1, e1, v1, id1)

      @pl.when(j < npair - 1)
      def _():
        idx_start(i0 + 3, is1, id1, smi1)

      return carry

    lax.fori_loop(0, npair, pair, 0)
    plsc.subcore_barrier()
    _readout(acc, out_h, c, s)

  bufset = [
      pltpu.VMEM((CKA,), jnp.int32),
      pltpu.VMEM((CKA,), jnp.int32),
      pltpu.VMEM((CKA, 8 * DIN), jnp.bfloat16),
      pltpu.VMEM((CKA, 16), jnp.float32),
      pltpu.VMEM((CKA, 16), jnp.float32),
  ]
  return pl.kernel(
      body,
      out_type=jax.ShapeDtypeStruct((NC * N, 2 * DIN), jnp.bfloat16),
      mesh=_sc_mesh(),
      compiler_params=pltpu.CompilerParams(use_tc_tiling_on_sc=False),
      scratch_types=bufset + bufset + [
          pltpu.SemaphoreType.DMA,
          pltpu.SemaphoreType.DMA,
          pltpu.SemaphoreType.DMA,
          pltpu.SemaphoreType.DMA,
          pltpu.VMEM((CKA, 2 * DIN), jnp.bfloat16),
          pltpu.VMEM_SHARED((NA, 2 * DIN), jnp.bfloat16),
      ],
  )(hh8, exw, r8, src, dst, zeros_h)


def _relu(v):
  return jnp.maximum(v, 0.0)


def _k1(degp0, degp1, x):
  def body(d0, d1, xr, xp, dv):
    deg = d0[:, :1] + d1[:, :1] + 1.0
    dinv = lax.rsqrt(deg)
    dvb = jnp.broadcast_to(dinv, (BN, DIN))
    dv[...] = dvb
    xp[...] = xr[...] * dvb

  bs16 = pl.BlockSpec((BN, 16), lambda i: (i, 0))
  bs128 = pl.BlockSpec((BN, DIN), lambda i: (i, 0))
  return pl.pallas_call(
      body,
      grid=(NB,),
      in_specs=[bs16, bs16, bs128],
      out_specs=[bs128, bs128],
      out_shape=[
          jax.ShapeDtypeStruct((N, DIN), jnp.float32),
          jax.ShapeDtypeStruct((N, DIN), jnp.float32),
      ],
  )(degp0, degp1, x)


def _k2(a0, a1, xp, dv, W1, b1):
  def body(a0r, a1r, xpr, dvr, w, b, lo, hi, h1pb):
    pre = dvr[...] * (a0r[...] + a1r[...] + xpr[...])
    h = _relu(jnp.dot(pre, w[...], preferred_element_type=jnp.float32) + b[...])
    h1p = h * dvr[:, :1]
    lo[...] = h1p[:, :DIN]
    hi[...] = h1p[:, DIN:]
    h1pb[...] = h1p.astype(jnp.bfloat16)

  bs128 = pl.BlockSpec((BN, DIN), lambda i: (i, 0))
  return pl.pallas_call(
      body,
      grid=(NB,),
      in_specs=[bs128, bs128, bs128, bs128,
                pl.BlockSpec((DIN, DH), lambda i: (0, 0)),
                pl.BlockSpec((1, DH), lambda i: (0, 0))],
      out_specs=[bs128, bs128, pl.BlockSpec((BN, DH), lambda i: (i, 0))],
      out_shape=[
          jax.ShapeDtypeStruct((N, DIN), jnp.float32),
          jax.ShapeDtypeStruct((N, DIN), jnp.float32),
          jax.ShapeDtypeStruct((N, DH), jnp.bfloat16),
      ],
  )(a0, a1, xp, dv, W1, b1)


def _k3(ag0f, ag1f, lo, hi, dv, W2, b2, W3, As, Ad):
  def body(ag0, ag1, lor, hir, dvr, w2, b2r, w3, asr, adr,
           hh8lo, hh8hi, hhb, als, ald):
    h1p = jnp.concatenate([lor[...], hir[...]], axis=1)
    agg = ag0[...].astype(jnp.float32) + ag1[...].astype(jnp.float32)
    pre = dvr[:, :1] * (agg + h1p)
    h2 = _relu(jnp.dot(pre, w2[...], preferred_element_type=jnp.float32)
               + b2r[...])
    hh = jnp.dot(h2, w3[...], preferred_element_type=jnp.float32)
    als[...] = jnp.dot(hh, asr[...], preferred_element_type=jnp.float32)
    ald[...] = jnp.dot(hh, adr[...], preferred_element_type=jnp.float32)
    hh8lo[...] = jnp.concatenate(
        [hh[:, 0:128], hh[:, 256:384], hh[:, 512:640], hh[:, 768:896]], axis=1)
    hh8hi[...] = jnp.concatenate(
        [hh[:, 128:256], hh[:, 384:512], hh[:, 640:768], hh[:, 896:1024]],
        axis=1)
    hhb[...] = hh.astype(jnp.bfloat16)

  bs128 = pl.BlockSpec((BN, DIN), lambda i: (i, 0))
  bs256b = pl.BlockSpec((BN, 2 * DIN), lambda i: (i, 0))
  bs512 = pl.BlockSpec((BN, 4 * DIN), lambda i: (i, 0))
  bs1024 = pl.BlockSpec((BN, 8 * DIN), lambda i: (i, 0))
  bs16 = pl.BlockSpec((BN, 16), lambda i: (i, 0))
  return pl.pallas_call(
      body,
      grid=(NB,),
      in_specs=[bs256b, bs256b, bs128, bs128, bs128,
                pl.BlockSpec((DH, DH), lambda i: (0, 0)),
                pl.BlockSpec((1, DH), lambda i: (0, 0)),
                pl.BlockSpec((DH, H * DH), lambda i: (0, 0)),
                pl.BlockSpec((H * DH, 16), lambda i: (0, 0)),
                pl.BlockSpec((H * DH, 16), lambda i: (0, 0))],
      out_specs=[bs512, bs512, bs1024, bs16, bs16],
      out_shape=[
          jax.ShapeDtypeStruct((N, 4 * DIN), jnp.float32),
          jax.ShapeDtypeStruct((N, 4 * DIN), jnp.float32),
          jax.ShapeDtypeStruct((N, 8 * DIN), jnp.bfloat16),
          jax.ShapeDtypeStruct((N, 16), jnp.float32),
          jax.ShapeDtypeStruct((N, 16), jnp.float32),
      ],
  )(ag0f, ag1f, lo, hi, dv, W2, b2, W3, As, Ad)


def _k3b(als, ald):
  def body(alsr, aldr, ms, md, cout):
    i = pl.program_id(0)

    @pl.when(i == 0)
    def _():
      ms[...] = jnp.full((1, 16), -1e30, jnp.float32)
      md[...] = jnp.full((1, 16), -1e30, jnp.float32)

    ms[...] = jnp.maximum(ms[...], jnp.max(alsr[...], axis=0, keepdims=True))
    md[...] = jnp.maximum(md[...], jnp.max(aldr[...], axis=0, keepdims=True))

    @pl.when(i == NB - 1)
    def _():
      a = ms[...]
      b = md[...]
      cout[...] = jnp.maximum(a, 0.2 * a) + jnp.maximum(b, 0.2 * b)

  bs16 = pl.BlockSpec((BN, 16), lambda i: (i, 0))
  os = pl.BlockSpec((1, 16), lambda i: (0, 0))
  return pl.pallas_call(
      body,
      grid=(NB,),
      in_specs=[bs16, bs16],
      out_specs=[os, os, os],
      out_shape=[jax.ShapeDtypeStruct((1, 16), jnp.float32)] * 3,
  )(als, ald)


def _k4(s0, s1, als, ald, C):
  def body(s0r, s1r, alsr, aldr, cr, r8, ws8):
    t = alsr[...] + aldr[...]
    t = jnp.maximum(t, 0.2 * t) - cr[...]
    exs = jnp.exp(t)
    stot = s0r[...] + s1r[...] + exs
    lane = lax.broadcasted_iota(jnp.int32, (BN, 16), 1)
    r = jnp.where(lane < H, 1.0 / (stot + 1e-16), 0.0)
    r8[...] = r
    ws8[...] = exs * r

  bs16 = pl.BlockSpec((BN, 16), lambda i: (i, 0))
  return pl.pallas_call(
      body,
      grid=(NB,),
      in_specs=[bs16, bs16, bs16, bs16,
                pl.BlockSpec((1, 16), lambda i: (0, 0))],
      out_specs=[bs16, bs16],
      out_shape=[jax.ShapeDtypeStruct((N, 16), jnp.float32)] * 2,
  )(s0, s1, als, ald, C)


def _k5(ag0, ag1, hlo, hhi, r8, ws8, b3f, batchT):
  def body(a0r, a1r, hlor, hhir, r8r, ws8r, b3r, btr, pool, cnt):
    i = pl.program_id(0)
    agg = a0r[...].astype(jnp.float32) + a1r[...].astype(jnp.float32)
    ws = ws8r[...]
    hlo_v = hlor[...]
    hhi_v = hhir[...]
    self_lo = ws[:, 0:1] * hlo_v[:, 0:128]
    self_hi = ws[:, 0:1] * hhi_v[:, 0:128]
    for h in range(1, H):
      self_lo = self_lo + ws[:, h:h + 1] * hlo_v[:, h * 128:(h + 1) * 128]
      self_hi = self_hi + ws[:, h:h + 1] * hhi_v[:, h * 128:(h + 1) * 128]
    m = 0.25 * (agg + jnp.concatenate([self_lo, self_hi], axis=1))
    h3 = _relu(m + b3r[...])
    mask = (lax.broadcasted_iota(jnp.int32, (BN, G), 1) == btr[...])
    mf = jnp.where(mask, 1.0, 0.0)

    @pl.when(i == 0)
    def _():
      pool[...] = jnp.zeros((G, DH), jnp.float32)
      cnt[...] = jnp.zeros((G, 8), jnp.float32)

    dn = (((0,), (0,)), ((), ()))
    pool[...] += lax.dot_general(mf, h3, dn,
                                 preferred_element_type=jnp.float32)
    cnt[...] += lax.dot_general(mf, jnp.ones((BN, 8), jnp.float32), dn,
                                preferred_element_type=jnp.float32)

  bs256b = pl.BlockSpec((BN, 2 * DIN), lambda i: (i, 0))
  bs512 = pl.BlockSpec((BN, 4 * DIN), lambda i: (i, 0))
  bs16 = pl.BlockSpec((BN, 16), lambda i: (i, 0))
  return pl.pallas_call(
      body,
      grid=(NB,),
      in_specs=[bs256b, bs256b, bs512, bs512, bs16, bs16,
                pl.BlockSpec((1, DH), lambda i: (0, 0)),
                pl.BlockSpec((BN, 1), lambda i: (i, 0))],
      out_specs=[pl.BlockSpec((G, DH), lambda i: (0, 0)),
                 pl.BlockSpec((G, 8), lambda i: (0, 0))],
      out_shape=[
          jax.ShapeDtypeStruct((G, DH), jnp.float32),
          jax.ShapeDtypeStruct((G, 8), jnp.float32),
      ],
  )(ag0, ag1, hlo, hhi, r8, ws8, b3f, batchT)


def _k6(pool, cnt, W4, b4):
  def body(poolr, cntr, w4, b4r, out):
    gr = poolr[...] / jnp.maximum(cntr[:, :1], 1.0)
    out[...] = jnp.dot(gr, w4[...], preferred_element_type=jnp.float32) + b4r[...]

  return pl.pallas_call(
      body,
      in_specs=[pl.BlockSpec((G, DH), lambda: (0, 0)),
                pl.BlockSpec((G, 8), lambda: (0, 0)),
                pl.BlockSpec((DH, 4), lambda: (0, 0)),
                pl.BlockSpec((1, 4), lambda: (0, 0))],
      out_specs=pl.BlockSpec((G, 4), lambda: (0, 0)),
      out_shape=jax.ShapeDtypeStruct((G, 4), jnp.float32),
  )(pool, cnt, W4, b4)


def kernel(x, edge_index, batch, W1, b1, W2, b2, W3, a_src, a_dst, b3,
           We, be, Wm, bm, Wb, bb, Wp, bp):
  npad = EP - E
  src = jnp.concatenate([edge_index[0], jnp.zeros((npad,), jnp.int32)])
  dst = jnp.concatenate([edge_index[1], jnp.full((npad,), N, jnp.int32)])

  zeros16 = jnp.zeros((NPS, 16), jnp.float32)
  zeros128 = jnp.zeros((NPS, DIN), jnp.float32)
  ones_ck = jnp.ones((CK, 16), jnp.float32)

  As = jnp.zeros((H * DH, 16), jnp.float32)
  Ad = jnp.zeros((H * DH, 16), jnp.float32)
  for h in range(H):
    As = As.at[h * DH:(h + 1) * DH, h].set(a_src[h])
    Ad = Ad.at[h * DH:(h + 1) * DH, h].set(a_dst[h])

  W4 = jnp.concatenate([We, Wm, Wb, Wp], axis=1)
  b4 = jnp.concatenate([be, bm, bb, bp]).reshape(1, 4)

  degp = _sc_count(dst, ones_ck, zeros16)
  xp, dv = _k1(degp[:N], degp[N:], x)
  a1p = _sc_seg_rows(xp, src, dst, zeros128)
  h1plo, h1phi, h1pb = _k2(a1p[:N], a1p[N:], xp, dv, W1, b1.reshape(1, DH))

  zeros256b = jnp.zeros((NPS, 2 * DIN), jnp.bfloat16)
  a2 = _sc_seg_rows_b(h1pb, src, dst, zeros256b)
  hh8lo, hh8hi, hhb, als, ald = _k3(
      a2[:N], a2[N:], h1plo, h1phi, dv, W2, b2.reshape(1, DH), W3, As, Ad)

  _, _, C = _k3b(als, ald)
  pad8 = jnp.zeros((NA - N, 16), jnp.float32)
  ex, sp = _sc_gat_logits(jnp.concatenate([als, pad8]),
                          jnp.concatenate([ald, pad8]),
                          C.reshape(16), src, dst, zeros16)
  r8, ws8 = _k4(sp[:N], sp[N:], als, ald, C)
  r8p = jnp.concatenate([r8, pad8])
  a3 = _sc_gat_agg(hhb, ex, r8p, src, dst, zeros256b)

  pool, cnt = _k5(a3[:N], a3[N:], hh8lo, hh8hi,
                  r8, ws8, b3.reshape(1, DH), batch.reshape(N, 1))
  return _k6(pool, cnt, W4, b4)


# speedup vs baseline: 1.2182x; 1.2182x over previous
import functools

import jax
import jax.numpy as jnp
from jax import lax
from jax.experimental import pallas as pl
from jax.experimental.pallas import tpu as pltpu
from jax.experimental.pallas import tpu_sc as plsc

N = 10000
E = 320000
DIN = 128
DH = 256
H = 4
G = 64

NC = 2
NS = 16
NW = NC * NS
EP = 327680
EPW = EP // NW
CK = 128
NCH = EPW // CK
CKA = 32
NCHA = EPW // CKA
NA = N + 8
NPS = 624
TBASE = NS * NPS
ZTAIL = NA - TBASE
RTAIL = N - TBASE

BN = 400
NB = N // BN


def _sc_mesh():
  return plsc.VectorSubcoreMesh(core_axis_name="c", subcore_axis_name="s",
                                num_cores=NC, num_subcores=NS)


def _wid():
  return lax.axis_index("c") * NS + lax.axis_index("s")


def _zero_acc(zeros_hh, acc, s):
  pltpu.sync_copy(zeros_hh, acc.at[pl.ds(s * NPS, NPS)])

  @pl.when(s == 0)
  def _():
    pltpu.sync_copy(zeros_hh.at[pl.ds(0, ZTAIL)], acc.at[pl.ds(TBASE, ZTAIL)])


def _readout(acc, out_h, c, s):
  pltpu.sync_copy(acc.at[pl.ds(s * NPS, NPS)],
                  out_h.at[pl.ds(c * N + s * NPS, NPS)])

  @pl.when(s == 0)
  def _():
    pltpu.sync_copy(acc.at[pl.ds(TBASE, RTAIL)],
                    out_h.at[pl.ds(c * N + TBASE, RTAIL)])


def _sc_count(dst, ones_h, zeros_h):
  def body(dst_h, ones_hh, zeros_hh, out_h, id0, id1, ones_v, smi0, smi1, acc):
    c = lax.axis_index("c")
    s = lax.axis_index("s")
    wid = c * NS + s
    ebase = wid * EPW
    _zero_acc(zeros_hh, acc, s)
    pltpu.sync_copy(ones_hh, ones_v)
    plsc.subcore_barrier()

    def idx_start(i, idb, smi):
      pltpu.async_copy(dst_h.at[pl.ds(ebase + i * CK, CK)], idb, smi)

    def idx_wait(i, idb, smi):
      pltpu.make_async_copy(dst_h.at[pl.ds(ebase + i * CK, CK)],
                            idb, smi).wait()

    idx_start(0, id0, smi0)
    idx_start(1, id1, smi1)
    npair = NCH // 2

    def pair(j, carry):
      i0 = 2 * j
      idx_wait(i0, id0, smi0)
      pltpu.sync_copy(ones_v, acc.at[id0], add=True)

      @pl.when(j < npair - 1)
      def _():
        idx_start(i0 + 2, id0, smi0)

      idx_wait(i0 + 1, id1, smi1)
      pltpu.sync_copy(ones_v, acc.at[id1], add=True)

      @pl.when(j < npair - 1)
      def _():
        idx_start(i0 + 3, id1, smi1)

      return carry

    lax.fori_loop(0, npair, pair, 0)
    plsc.subcore_barrier()
    _readout(acc, out_h, c, s)

  return pl.kernel(
      body,
      out_type=jax.ShapeDtypeStruct((NC * N, 16), jnp.float32),
      mesh=_sc_mesh(),
      compiler_params=pltpu.CompilerParams(use_tc_tiling_on_sc=False),
      scratch_types=[
          pltpu.VMEM((CK,), jnp.int32),
          pltpu.VMEM((CK,), jnp.int32),
          pltpu.VMEM((CK, 16), jnp.float32),
          pltpu.SemaphoreType.DMA,
          pltpu.SemaphoreType.DMA,
          pltpu.VMEM_SHARED((NA, 16), jnp.float32),
      ],
  )(dst, ones_h, zeros_h)


def _sc_seg_rows(table, src, dst, zeros_h):
  def body(tab_h, src_h, dst_h, zeros_hh, out_h,
           is0, id0, r0, is1, id1, r1, smi0, smi1, smg0, smg1, acc):
    c = lax.axis_index("c")
    s = lax.axis_index("s")
    wid = c * NS + s
    ebase = wid * EPW
    _zero_acc(zeros_hh, acc, s)
    plsc.subcore_barrier()

    def idx_start(i, isb, idb, smi):
      b = ebase + i * CK
      pltpu.async_copy(src_h.at[pl.ds(b, CK)], isb, smi)
      pltpu.async_copy(dst_h.at[pl.ds(b, CK)], idb, smi)

    def idx_wait(i, isb, idb, smi):
      b = ebase + i * CK
      pltpu.make_async_copy(src_h.at[pl.ds(b, CK)], isb, smi).wait()
      pltpu.make_async_copy(dst_h.at[pl.ds(b, CK)], idb, smi).wait()

    pltpu.sync_copy(src_h.at[pl.ds(ebase, CK)], is0)
    pltpu.sync_copy(dst_h.at[pl.ds(ebase, CK)], id0)
    pltpu.async_copy(tab_h.at[is0], r0, smg0)
    idx_start(1, is1, id1, smi1)

    npair = NCH // 2

    def pair(j, carry):
      i0 = 2 * j
      idx_wait(i0 + 1, is1, id1, smi1)
      pltpu.async_copy(tab_h.at[is1], r1, smg1)
      pltpu.make_async_copy(tab_h.at[is0], r0, smg0).wait()
      pltpu.sync_copy(r0, acc.at[id0], add=True)

      @pl.when(j < npair - 1)
      def _():
        idx_start(i0 + 2, is0, id0, smi0)

      @pl.when(j < npair - 1)
      def _():
        idx_wait(i0 + 2, is0, id0, smi0)
        pltpu.async_copy(tab_h.at[is0], r0, smg0)

      pltpu.make_async_copy(tab_h.at[is1], r1, smg1).wait()
      pltpu.sync_copy(r1, acc.at[id1], add=True)

      @pl.when(j < npair - 1)
      def _():
        idx_start(i0 + 3, is1, id1, smi1)

      return carry

    lax.fori_loop(0, npair, pair, 0)
    plsc.subcore_barrier()
    _readout(acc, out_h, c, s)

  return pl.kernel(
      body,
      out_type=jax.ShapeDtypeStruct((NC * N, DIN), jnp.float32),
      mesh=_sc_mesh(),
      compiler_params=pltpu.CompilerParams(use_tc_tiling_on_sc=False),
      scratch_types=[
          pltpu.VMEM((CK,), jnp.int32),
          pltpu.VMEM((CK,), jnp.int32),
          pltpu.VMEM((CK, DIN), jnp.float32),
          pltpu.VMEM((CK,), jnp.int32),
          pltpu.VMEM((CK,), jnp.int32),
          pltpu.VMEM((CK, DIN), jnp.float32),
          pltpu.SemaphoreType.DMA,
          pltpu.SemaphoreType.DMA,
          pltpu.SemaphoreType.DMA,
          pltpu.SemaphoreType.DMA,
          pltpu.VMEM_SHARED((NA, DIN), jnp.float32),
      ],
  )(table, src, dst, zeros_h)


def _sc_seg_rows_b(table, src, dst, zeros_h):
  def body(tab_h, src_h, dst_h, zeros_hh, out_h,
           is0, id0, r0, is1, id1, r1, smi0, smi1, smg0, smg1, acc):
    c = lax.axis_index("c")
    s = lax.axis_index("s")
    wid = c * NS + s
    ebase = wid * EPW
    _zero_acc(zeros_hh, acc, s)
    plsc.subcore_barrier()

    def idx_start(i, isb, idb, smi):
      b = ebase + i * CK
      pltpu.async_copy(src_h.at[pl.ds(b, CK)], isb, smi)
      pltpu.async_copy(dst_h.at[pl.ds(b, CK)], idb, smi)

    def idx_wait(i, isb, idb, smi):
      b = ebase + i * CK
      pltpu.make_async_copy(src_h.at[pl.ds(b, CK)], isb, smi).wait()
      pltpu.make_async_copy(dst_h.at[pl.ds(b, CK)], idb, smi).wait()

    pltpu.sync_copy(src_h.at[pl.ds(ebase, CK)], is0)
    pltpu.sync_copy(dst_h.at[pl.ds(ebase, CK)], id0)
    pltpu.async_copy(tab_h.at[is0], r0, smg0)
    idx_start(1, is1, id1, smi1)

    npair = NCH // 2

    def pair(j, carry):
      i0 = 2 * j
      idx_wait(i0 + 1, is1, id1, smi1)
      pltpu.async_copy(tab_h.at[is1], r1, smg1)
      pltpu.make_async_copy(tab_h.at[is0], r0, smg0).wait()
      pltpu.sync_copy(r0, acc.at[id0], add=True)

      @pl.when(j < npair - 1)
      def _():
        idx_start(i0 + 2, is0, id0, smi0)

      @pl.when(j < npair - 1)
      def _():
        idx_wait(i0 + 2, is0, id0, smi0)
        pltpu.async_copy(tab_h.at[is0], r0, smg0)

      pltpu.make_async_copy(tab_h.at[is1], r1, smg1).wait()
      pltpu.sync_copy(r1, acc.at[id1], add=True)

      @pl.when(j < npair - 1)
      def _():
        idx_start(i0 + 3, is1, id1, smi1)

      return carry

    lax.fori_loop(0, npair, pair, 0)
    plsc.subcore_barrier()
    _readout(acc, out_h, c, s)

  return pl.kernel(
      body,
      out_type=jax.ShapeDtypeStruct((NC * N, 2 * DIN), jnp.bfloat16),
      mesh=_sc_mesh(),
      compiler_params=pltpu.CompilerParams(use_tc_tiling_on_sc=False),
      scratch_types=[
          pltpu.VMEM((CK,), jnp.int32),
          pltpu.VMEM((CK,), jnp.int32),
          pltpu.VMEM((CK, 2 * DIN), jnp.bfloat16),
          pltpu.VMEM((CK,), jnp.int32),
          pltpu.VMEM((CK,), jnp.int32),
          pltpu.VMEM((CK, 2 * DIN), jnp.bfloat16),
          pltpu.SemaphoreType.DMA,
          pltpu.SemaphoreType.DMA,
          pltpu.SemaphoreType.DMA,
          pltpu.SemaphoreType.DMA,
          pltpu.VMEM_SHARED((NA, 2 * DIN), jnp.bfloat16),
      ],
  )(table, src, dst, zeros_h)


def _sc_gat_logits(als8, ald8, cvec, src, dst, zeros_h):
  def body(als_h, ald_h, c_h, src_h, dst_h, zeros_hh,
           ex_h, out_h, idx_s, idx_d, asv, adv, exb, cv, acc):
    c = lax.axis_index("c")
    s = lax.axis_index("s")
    wid = c * NS + s
    _zero_acc(zeros_hh, acc, s)
    pltpu.sync_copy(c_h, cv)
    plsc.subcore_barrier()

    def chunk(i, carry):
      base = wid * EPW + i * CK
      pltpu.sync_copy(src_h.at[pl.ds(base, CK)], idx_s)
      pltpu.sync_copy(dst_h.at[pl.ds(base, CK)], idx_d)
      pltpu.sync_copy(als_h.at[idx_s], asv)
      pltpu.sync_copy(ald_h.at[idx_d], adv)
      cvv = cv[...]

      def epair(p, carry2):
        j0 = 2 * p
        j1 = 2 * p + 1
        t0 = asv[j0, :] + adv[j0, :]
        t1 = asv[j1, :] + adv[j1, :]
        t0 = jnp.maximum(t0, 0.2 * t0) - cvv
        t1 = jnp.maximum(t1, 0.2 * t1) - cvv
        exb[j0, :] = jnp.exp(t0)
        exb[j1, :] = jnp.exp(t1)
        return carry2

      lax.fori_loop(0, CK // 2, epair, 0)
      pltpu.sync_copy(exb, ex_h.at[pl.ds(base, CK)])
      pltpu.sync_copy(exb, acc.at[idx_d], add=True)
      return carry

    lax.fori_loop(0, NCH, chunk, 0)
    plsc.subcore_barrier()
    _readout(acc, out_h, c, s)

  return pl.kernel(
      body,
      out_type=[
          jax.ShapeDtypeStruct((EP, 16), jnp.float32),
          jax.ShapeDtypeStruct((NC * N, 16), jnp.float32),
      ],
      mesh=_sc_mesh(),
      compiler_params=pltpu.CompilerParams(use_tc_tiling_on_sc=False),
      scratch_types=[
          pltpu.VMEM((CK,), jnp.int32),
          pltpu.VMEM((CK,), jnp.int32),
          pltpu.VMEM((CK, 16), jnp.float32),
          pltpu.VMEM((CK, 16), jnp.float32),
          pltpu.VMEM((CK, 16), jnp.float32),
          pltpu.VMEM((16,), jnp.float32),
          pltpu.VMEM_SHARED((NA, 16), jnp.float32),
      ],
  )(als8, ald8, cvec, src, dst, zeros_h)


def _sc_gat_agg(hh8, exw, r8, src, dst, zeros_h):
  def body(hh_h, ex_h, r_h, src_h, dst_h, zeros_hh, out_h,
           is0, id0, r0, e0, v0, is1, id1, r1, e1, v1,
           smi0, smi1, smg0, smg1, y, acc):
    c = lax.axis_index("c")
    s = lax.axis_index("s")
    wid = c * NS + s
    ebase = wid * EPW
    _zero_acc(zeros_hh, acc, s)
    plsc.subcore_barrier()

    def idx_start(i, isb, idb, smi):
      b = ebase + i * CKA
      pltpu.async_copy(src_h.at[pl.ds(b, CKA)], isb, smi)
      pltpu.async_copy(dst_h.at[pl.ds(b, CKA)], idb, smi)

    def idx_wait(i, isb, idb, smi):
      b = ebase + i * CKA
      pltpu.make_async_copy(src_h.at[pl.ds(b, CKA)], isb, smi).wait()
      pltpu.make_async_copy(dst_h.at[pl.ds(b, CKA)], idb, smi).wait()

    def fetch_start(i, isb, idb, rb, eb, vb, smg):
      b = ebase + i * CKA
      pltpu.async_copy(hh_h.at[isb], rb, smg)
      pltpu.async_copy(ex_h.at[pl.ds(b, CKA)], eb, smg)
      pltpu.async_copy(r_h.at[idb], vb, smg)

    def fetch_wait(i, isb, idb, rb, eb, vb, smg):
      b = ebase + i * CKA
      pltpu.make_async_copy(hh_h.at[isb], rb, smg).wait()
      pltpu.make_async_copy(ex_h.at[pl.ds(b, CKA)], eb, smg).wait()
      pltpu.make_async_copy(r_h.at[idb], vb, smg).wait()

    def compute_scatter(rb, eb, vb, idb):
      def epair(p, carry2):
        j0 = 2 * p
        j1 = 2 * p + 1
        aa = eb[j0, :] * vb[j0, :]
        ab = eb[j1, :] * vb[j1, :]

        def splat(av, h):
          return jnp.full((16,), av[h], jnp.float32).astype(jnp.bfloat16)

        w = [splat(aa, h) for h in range(4)] + [splat(ab, h) for h in range(4)]
        for cb in range(16):
          va = (w[0] * rb[j0, pl.ds(cb * 16, 16)]
                + w[1] * rb[j0, pl.ds(256 + cb * 16, 16)]
                + w[2] * rb[j0, pl.ds(512 + cb * 16, 16)]
                + w[3] * rb[j0, pl.ds(768 + cb * 16, 16)])
          vb2 = (w[4] * rb[j1, pl.ds(cb * 16, 16)]
                 + w[5] * rb[j1, pl.ds(256 + cb * 16, 16)]
                 + w[6] * rb[j1, pl.ds(512 + cb * 16, 16)]
                 + w[7] * rb[j1, pl.ds(768 + cb * 16, 16)])
          y[j0, pl.ds(cb * 16, 16)] = va
          y[j1, pl.ds(cb * 16, 16)] = vb2
        return carry2

      lax.fori_loop(0, CKA // 2, epair, 0)
      pltpu.sync_copy(y, acc.at[idb], add=True)

    pltpu.sync_copy(src_h.at[pl.ds(ebase, CKA)], is0)
    pltpu.sync_copy(dst_h.at[pl.ds(ebase, CKA)], id0)
    fetch_start(0, is0, id0, r0, e0, v0, smg0)
    idx_start(1, is1, id1, smi1)

    npair = NCHA // 2

    def pair(j, carry):
      i0 = 2 * j
      idx_wait(i0 + 1, is1, id1, smi1)
      fetch_start(i0 + 1, is1, id1, r1, e1, v1, smg1)
      fetch_wait(i0, is0, id0, r0, e0, v0, smg0)
      compute_scatter(r0, e0, v0, id0)

      @pl.when(j < npair - 1)
      def _():
        idx_start(i0 + 2, is0, id0, smi0)
        idx_wait(i0 + 2, is0, id0, smi0)
        fetch_start(i0 + 2, is0, id0, r0, e0, v0, smg0)

      fetch_wait(i0 + 1, is1, id1, r1, e1, v1, smg1)
      compute_scatter(r1, e1, v1, id1)

      @pl.when(j < npair - 1)
      def _():
        idx_start(i0 + 3, is1, id1, smi1)

      return carry

    lax.fori_loop(0, npair, pair, 0)
    plsc.subcore_barrier()
    _readout(acc, out_h, c, s)

  bufset = [
      pltpu.VMEM((CKA,), jnp.int32),
      pltpu.VMEM((CKA,), jnp.int32),
      pltpu.VMEM((CKA, 8 * DIN), jnp.bfloat16),
      pltpu.VMEM((CKA, 16), jnp.float32),
      pltpu.VMEM((CKA, 16), jnp.float32),
  ]
  return pl.kernel(
      body,
      out_type=jax.ShapeDtypeStruct((NC * N, 2 * DIN), jnp.bfloat16),
      mesh=_sc_mesh(),
      compiler_params=pltpu.CompilerParams(use_tc_tiling_on_sc=False),
      scratch_types=bufset + bufset + [
          pltpu.SemaphoreType.DMA,
          pltpu.SemaphoreType.DMA,
          pltpu.SemaphoreType.DMA,
          pltpu.SemaphoreType.DMA,
          pltpu.VMEM((CKA, 2 * DIN), jnp.bfloat16),
          pltpu.VMEM_SHARED((NA, 2 * DIN), jnp.bfloat16),
      ],
  )(hh8, exw, r8, src, dst, zeros_h)


def _relu(v):
  return jnp.maximum(v, 0.0)


def _k1(degp0, degp1, x):
  def body(d0, d1, xr, xp, dv):
    deg = d0[:, :1] + d1[:, :1] + 1.0
    dinv = lax.rsqrt(deg)
    dvb = jnp.broadcast_to(dinv, (BN, DIN))
    dv[...] = dvb
    xp[...] = xr[...] * dvb

  bs16 = pl.BlockSpec((BN, 16), lambda i: (i, 0))
  bs128 = pl.BlockSpec((BN, DIN), lambda i: (i, 0))
  return pl.pallas_call(
      body,
      grid=(NB,),
      in_specs=[bs16, bs16, bs128],
      out_specs=[bs128, bs128],
      out_shape=[
          jax.ShapeDtypeStruct((N, DIN), jnp.float32),
          jax.ShapeDtypeStruct((N, DIN), jnp.float32),
      ],
  )(degp0, degp1, x)


def _k2(a0, a1, xp, dv, W1, b1):
  def body(a0r, a1r, xpr, dvr, w, b, lo, hi, h1pb):
    pre = dvr[...] * (a0r[...] + a1r[...] + xpr[...])
    h = _relu(jnp.dot(pre, w[...], preferred_element_type=jnp.float32) + b[...])
    h1p = h * dvr[:, :1]
    lo[...] = h1p[:, :DIN]
    hi[...] = h1p[:, DIN:]
    h1pb[...] = h1p.astype(jnp.bfloat16)

  bs128 = pl.BlockSpec((BN, DIN), lambda i: (i, 0))
  return pl.pallas_call(
      body,
      grid=(NB,),
      in_specs=[bs128, bs128, bs128, bs128,
                pl.BlockSpec((DIN, DH), lambda i: (0, 0)),
                pl.BlockSpec((1, DH), lambda i: (0, 0))],
      out_specs=[bs128, bs128, pl.BlockSpec((BN, DH), lambda i: (i, 0))],
      out_shape=[
          jax.ShapeDtypeStruct((N, DIN), jnp.float32),
          jax.ShapeDtypeStruct((N, DIN), jnp.float32),
          jax.ShapeDtypeStruct((N, DH), jnp.bfloat16),
      ],
  )(a0, a1, xp, dv, W1, b1)


def _k3(ag0f, ag1f, lo, hi, dv, W2, b2, W3, As, Ad):
  def body(ag0, ag1, lor, hir, dvr, w2, b2r, w3, asr, adr,
           hh8lo, hh8hi, hhb, als, ald):
    h1p = jnp.concatenate([lor[...], hir[...]], axis=1)
    agg = ag0[...].astype(jnp.float32) + ag1[...].astype(jnp.float32)
    pre = dvr[:, :1] * (agg + h1p)
    h2 = _relu(jnp.dot(pre, w2[...], preferred_element_type=jnp.float32)
               + b2r[...])
    hh = jnp.dot(h2, w3[...], preferred_element_type=jnp.float32)
    als[...] = jnp.dot(hh, asr[...], preferred_element_type=jnp.float32)
    ald[...] = jnp.dot(hh, adr[...], preferred_element_type=jnp.float32)
    hh8lo[...] = jnp.concatenate(
        [hh[:, 0:128], hh[:, 256:384], hh[:, 512:640], hh[:, 768:896]], axis=1)
    hh8hi[...] = jnp.concatenate(
        [hh[:, 128:256], hh[:, 384:512], hh[:, 640:768], hh[:, 896:1024]],
        axis=1)
    hhb[...] = hh.astype(jnp.bfloat16)

  bs128 = pl.BlockSpec((BN, DIN), lambda i: (i, 0))
  bs256b = pl.BlockSpec((BN, 2 * DIN), lambda i: (i, 0))
  bs512 = pl.BlockSpec((BN, 4 * DIN), lambda i: (i, 0))
  bs1024 = pl.BlockSpec((BN, 8 * DIN), lambda i: (i, 0))
  bs16 = pl.BlockSpec((BN, 16), lambda i: (i, 0))
  return pl.pallas_call(
      body,
      grid=(NB,),
      in_specs=[bs256b, bs256b, bs128, bs128, bs128,
                pl.BlockSpec((DH, DH), lambda i: (0, 0)),
                pl.BlockSpec((1, DH), lambda i: (0, 0)),
                pl.BlockSpec((DH, H * DH), lambda i: (0, 0)),
                pl.BlockSpec((H * DH, 16), lambda i: (0, 0)),
                pl.BlockSpec((H * DH, 16), lambda i: (0, 0))],
      out_specs=[bs512, bs512, bs1024, bs16, bs16],
      out_shape=[
          jax.ShapeDtypeStruct((N, 4 * DIN), jnp.float32),
          jax.ShapeDtypeStruct((N, 4 * DIN), jnp.float32),
          jax.ShapeDtypeStruct((N, 8 * DIN), jnp.bfloat16),
          jax.ShapeDtypeStruct((N, 16), jnp.float32),
          jax.ShapeDtypeStruct((N, 16), jnp.float32),
      ],
  )(ag0f, ag1f, lo, hi, dv, W2, b2, W3, As, Ad)


def _k3b(als, ald):
  def body(alsr, aldr, ms, md, cout):
    i = pl.program_id(0)

    @pl.when(i == 0)
    def _():
      ms[...] = jnp.full((1, 16), -1e30, jnp.float32)
      md[...] = jnp.full((1, 16), -1e30, jnp.float32)

    ms[...] = jnp.maximum(ms[...], jnp.max(alsr[...], axis=0, keepdims=True))
    md[...] = jnp.maximum(md[...], jnp.max(aldr[...], axis=0, keepdims=True))

    @pl.when(i == NB - 1)
    def _():
      a = ms[...]
      b = md[...]
      cout[...] = jnp.maximum(a, 0.2 * a) + jnp.maximum(b, 0.2 * b)

  bs16 = pl.BlockSpec((BN, 16), lambda i: (i, 0))
  os = pl.BlockSpec((1, 16), lambda i: (0, 0))
  return pl.pallas_call(
      body,
      grid=(NB,),
      in_specs=[bs16, bs16],
      out_specs=[os, os, os],
      out_shape=[jax.ShapeDtypeStruct((1, 16), jnp.float32)] * 3,
  )(als, ald)


def _k4(s0, s1, als, ald, C):
  def body(s0r, s1r, alsr, aldr, cr, r8, ws8):
    t = alsr[...] + aldr[...]
    t = jnp.maximum(t, 0.2 * t) - cr[...]
    exs = jnp.exp(t)
    stot = s0r[...] + s1r[...] + exs
    lane = lax.broadcasted_iota(jnp.int32, (BN, 16), 1)
    r = jnp.where(lane < H, 1.0 / (stot + 1e-16), 0.0)
    r8[...] = r
    ws8[...] = exs * r

  bs16 = pl.BlockSpec((BN, 16), lambda i: (i, 0))
  return pl.pallas_call(
      body,
      grid=(NB,),
      in_specs=[bs16, bs16, bs16, bs16,
                pl.BlockSpec((1, 16), lambda i: (0, 0))],
      out_specs=[bs16, bs16],
      out_shape=[jax.ShapeDtypeStruct((N, 16), jnp.float32)] * 2,
  )(s0, s1, als, ald, C)


def _k5(ag0, ag1, hlo, hhi, r8, ws8, b3f, batchT):
  def body(a0r, a1r, hlor, hhir, r8r, ws8r, b3r, btr, pool, cnt):
    i = pl.program_id(0)
    agg = a0r[...].astype(jnp.float32) + a1r[...].astype(jnp.float32)
    ws = ws8r[...]
    hlo_v = hlor[...]
    hhi_v = hhir[...]
    self_lo = ws[:, 0:1] * hlo_v[:, 0:128]
    self_hi = ws[:, 0:1] * hhi_v[:, 0:128]
    for h in range(1, H):
      self_lo = self_lo + ws[:, h:h + 1] * hlo_v[:, h * 128:(h + 1) * 128]
      self_hi = self_hi + ws[:, h:h + 1] * hhi_v[:, h * 128:(h + 1) * 128]
    m = 0.25 * (agg + jnp.concatenate([self_lo, self_hi], axis=1))
    h3 = _relu(m + b3r[...])
    mask = (lax.broadcasted_iota(jnp.int32, (BN, G), 1) == btr[...])
    mf = jnp.where(mask, 1.0, 0.0)

    @pl.when(i == 0)
    def _():
      pool[...] = jnp.zeros((G, DH), jnp.float32)
      cnt[...] = jnp.zeros((G, 8), jnp.float32)

    dn = (((0,), (0,)), ((), ()))
    pool[...] += lax.dot_general(mf, h3, dn,
                                 preferred_element_type=jnp.float32)
    cnt[...] += lax.dot_general(mf, jnp.ones((BN, 8), jnp.float32), dn,
                                preferred_element_type=jnp.float32)

  bs256b = pl.BlockSpec((BN, 2 * DIN), lambda i: (i, 0))
  bs512 = pl.BlockSpec((BN, 4 * DIN), lambda i: (i, 0))
  bs16 = pl.BlockSpec((BN, 16), lambda i: (i, 0))
  return pl.pallas_call(
      body,
      grid=(NB,),
      in_specs=[bs256b, bs256b, bs512, bs512, bs16, bs16,
                pl.BlockSpec((1, DH), lambda i: (0, 0)),
                pl.BlockSpec((BN, 1), lambda i: (i, 0))],
      out_specs=[pl.BlockSpec((G, DH), lambda i: (0, 0)),
                 pl.BlockSpec((G, 8), lambda i: (0, 0))],
      out_shape=[
          jax.ShapeDtypeStruct((G, DH), jnp.float32),
          jax.ShapeDtypeStruct((G, 8), jnp.float32),
      ],
  )(ag0, ag1, hlo, hhi, r8, ws8, b3f, batchT)


def _k6(pool, cnt, W4, b4):
  def body(poolr, cntr, w4, b4r, out):
    gr = poolr[...] / jnp.maximum(cntr[:, :1], 1.0)
    out[...] = jnp.dot(gr, w4[...], preferred_element_type=jnp.float32) + b4r[...]

  return pl.pallas_call(
      body,
      in_specs=[pl.BlockSpec((G, DH), lambda: (0, 0)),
                pl.BlockSpec((G, 8), lambda: (0, 0)),
                pl.BlockSpec((DH, 4), lambda: (0, 0)),
                pl.BlockSpec((1, 4), lambda: (0, 0))],
      out_specs=pl.BlockSpec((G, 4), lambda: (0, 0)),
      out_shape=jax.ShapeDtypeStruct((G, 4), jnp.float32),
  )(pool, cnt, W4, b4)


def kernel(x, edge_index, batch, W1, b1, W2, b2, W3, a_src, a_dst, b3,
           We, be, Wm, bm, Wb, bb, Wp, bp):
  npad = EP - E
  src = jnp.concatenate([edge_index[0], jnp.zeros((npad,), jnp.int32)])
  dst = jnp.concatenate([edge_index[1], jnp.full((npad,), N, jnp.int32)])

  zeros16 = jnp.zeros((NPS, 16), jnp.float32)
  zeros128 = jnp.zeros((NPS, DIN), jnp.float32)
  ones_ck = jnp.ones((CK, 16), jnp.float32)

  As = jnp.zeros((H * DH, 16), jnp.float32)
  Ad = jnp.zeros((H * DH, 16), jnp.float32)
  for h in range(H):
    As = As.at[h * DH:(h + 1) * DH, h].set(a_src[h])
    Ad = Ad.at[h * DH:(h + 1) * DH, h].set(a_dst[h])

  W4 = jnp.concatenate([We, Wm, Wb, Wp], axis=1)
  b4 = jnp.concatenate([be, bm, bb, bp]).reshape(1, 4)

  degp = _sc_count(dst, ones_ck, zeros16)
  xp, dv = _k1(degp[:N], degp[N:], x)
  a1p = _sc_seg_rows(xp, src, dst, zeros128)
  h1plo, h1phi, h1pb = _k2(a1p[:N], a1p[N:], xp, dv, W1, b1.reshape(1, DH))

  zeros256b = jnp.zeros((NPS, 2 * DIN), jnp.bfloat16)
  a2 = _sc_seg_rows_b(h1pb, src, dst, zeros256b)
  hh8lo, hh8hi, hhb, als, ald = _k3(
      a2[:N], a2[N:], h1plo, h1phi, dv, W2, b2.reshape(1, DH), W3, As, Ad)

  _, _, C = _k3b(als, ald)
  pad8 = jnp.zeros((NA - N, 16), jnp.float32)
  ex, sp = _sc_gat_logits(jnp.concatenate([als, pad8]),
                          jnp.concatenate([ald, pad8]),
                          C.reshape(16), src, dst, zeros16)
  r8, ws8 = _k4(sp[:N], sp[N:], als, ald, C)
  r8p = jnp.concatenate([r8, pad8])
  a3 = _sc_gat_agg(hhb, ex, r8p, src, dst, zeros256b)

  pool, cnt = _k5(a3[:N], a3[N:], hh8lo, hh8hi,
                  r8, ws8, b3.reshape(1, DH), batch.reshape(N, 1))
  return _k6(pool, cnt, W4, b4)


# speedup vs baseline: 1.3458x; 1.1048x over previous
import functools

import jax
import jax.numpy as jnp
from jax import lax
from jax.experimental import pallas as pl
from jax.experimental.pallas import tpu as pltpu
from jax.experimental.pallas import tpu_sc as plsc

N = 10000
E = 320000
DIN = 128
DH = 256
H = 4
G = 64

NC = 2
NS = 16
NW = NC * NS
EP = 327680
EPW = EP // NW
CK = 128
NCH = EPW // CK
CKC = 512
NCHC = EPW // CKC
CKA = 32
NCHA = EPW // CKA
NA = N + 8
NPS = 624
TBASE = NS * NPS
ZTAIL = NA - TBASE
RTAIL = N - TBASE

BN = 400
NB = N // BN


def _sc_mesh():
  return plsc.VectorSubcoreMesh(core_axis_name="c", subcore_axis_name="s",
                                num_cores=NC, num_subcores=NS)


def _wid():
  return lax.axis_index("c") * NS + lax.axis_index("s")


def _zero_acc(zeros_hh, acc, s):
  pltpu.sync_copy(zeros_hh, acc.at[pl.ds(s * NPS, NPS)])

  @pl.when(s == 0)
  def _():
    pltpu.sync_copy(zeros_hh.at[pl.ds(0, ZTAIL)], acc.at[pl.ds(TBASE, ZTAIL)])


def _readout(acc, out_h, c, s):
  pltpu.sync_copy(acc.at[pl.ds(s * NPS, NPS)],
                  out_h.at[pl.ds(c * N + s * NPS, NPS)])

  @pl.when(s == 0)
  def _():
    pltpu.sync_copy(acc.at[pl.ds(TBASE, RTAIL)],
                    out_h.at[pl.ds(c * N + TBASE, RTAIL)])


def _sc_count(dst, ones_h, zeros_h):
  def body(dst_h, ones_hh, zeros_hh, out_h, id0, id1, ones_v, smi0, smi1, acc):
    c = lax.axis_index("c")
    s = lax.axis_index("s")
    wid = c * NS + s
    ebase = wid * EPW
    _zero_acc(zeros_hh, acc, s)
    pltpu.sync_copy(ones_hh, ones_v)
    plsc.subcore_barrier()

    def idx_start(i, idb, smi):
      pltpu.async_copy(dst_h.at[pl.ds(ebase + i * CKC, CKC)], idb, smi)

    def idx_wait(i, idb, smi):
      pltpu.make_async_copy(dst_h.at[pl.ds(ebase + i * CKC, CKC)],
                            idb, smi).wait()

    idx_start(0, id0, smi0)
    idx_start(1, id1, smi1)
    npair = NCHC // 2

    def pair(j, carry):
      i0 = 2 * j
      idx_wait(i0, id0, smi0)
      pltpu.sync_copy(ones_v, acc.at[id0], add=True)

      @pl.when(j < npair - 1)
      def _():
        idx_start(i0 + 2, id0, smi0)

      idx_wait(i0 + 1, id1, smi1)
      pltpu.sync_copy(ones_v, acc.at[id1], add=True)

      @pl.when(j < npair - 1)
      def _():
        idx_start(i0 + 3, id1, smi1)

      return carry

    lax.fori_loop(0, npair, pair, 0)
    plsc.subcore_barrier()
    _readout(acc, out_h, c, s)

  return pl.kernel(
      body,
      out_type=jax.ShapeDtypeStruct((NC * N, 16), jnp.float32),
      mesh=_sc_mesh(),
      compiler_params=pltpu.CompilerParams(use_tc_tiling_on_sc=False),
      scratch_types=[
          pltpu.VMEM((CKC,), jnp.int32),
          pltpu.VMEM((CKC,), jnp.int32),
          pltpu.VMEM((CKC, 16), jnp.float32),
          pltpu.SemaphoreType.DMA,
          pltpu.SemaphoreType.DMA,
          pltpu.VMEM_SHARED((NA, 16), jnp.float32),
      ],
  )(dst, ones_h, zeros_h)


def _sc_seg_rows(table, src, dst, zeros_h):
  def body(tab_h, src_h, dst_h, zeros_hh, out_h,
           is0, id0, r0, is1, id1, r1, smi0, smi1, smg0, smg1, acc):
    c = lax.axis_index("c")
    s = lax.axis_index("s")
    wid = c * NS + s
    ebase = wid * EPW
    _zero_acc(zeros_hh, acc, s)
    plsc.subcore_barrier()

    def idx_start(i, isb, idb, smi):
      b = ebase + i * CK
      pltpu.async_copy(src_h.at[pl.ds(b, CK)], isb, smi)
      pltpu.async_copy(dst_h.at[pl.ds(b, CK)], idb, smi)

    def idx_wait(i, isb, idb, smi):
      b = ebase + i * CK
      pltpu.make_async_copy(src_h.at[pl.ds(b, CK)], isb, smi).wait()
      pltpu.make_async_copy(dst_h.at[pl.ds(b, CK)], idb, smi).wait()

    pltpu.sync_copy(src_h.at[pl.ds(ebase, CK)], is0)
    pltpu.sync_copy(dst_h.at[pl.ds(ebase, CK)], id0)
    pltpu.async_copy(tab_h.at[is0], r0, smg0)
    idx_start(1, is1, id1, smi1)

    npair = NCH // 2

    def pair(j, carry):
      i0 = 2 * j
      idx_wait(i0 + 1, is1, id1, smi1)
      pltpu.async_copy(tab_h.at[is1], r1, smg1)
      pltpu.make_async_copy(tab_h.at[is0], r0, smg0).wait()
      pltpu.sync_copy(r0, acc.at[id0], add=True)

      @pl.when(j < npair - 1)
      def _():
        idx_start(i0 + 2, is0, id0, smi0)

      @pl.when(j < npair - 1)
      def _():
        idx_wait(i0 + 2, is0, id0, smi0)
        pltpu.async_copy(tab_h.at[is0], r0, smg0)

      pltpu.make_async_copy(tab_h.at[is1], r1, smg1).wait()
      pltpu.sync_copy(r1, acc.at[id1], add=True)

      @pl.when(j < npair - 1)
      def _():
        idx_start(i0 + 3, is1, id1, smi1)

      return carry

    lax.fori_loop(0, npair, pair, 0)
    plsc.subcore_barrier()
    _readout(acc, out_h, c, s)

  return pl.kernel(
      body,
      out_type=jax.ShapeDtypeStruct((NC * N, DIN), jnp.float32),
      mesh=_sc_mesh(),
      compiler_params=pltpu.CompilerParams(use_tc_tiling_on_sc=False),
      scratch_types=[
          pltpu.VMEM((CK,), jnp.int32),
          pltpu.VMEM((CK,), jnp.int32),
          pltpu.VMEM((CK, DIN), jnp.float32),
          pltpu.VMEM((CK,), jnp.int32),
          pltpu.VMEM((CK,), jnp.int32),
          pltpu.VMEM((CK, DIN), jnp.float32),
          pltpu.SemaphoreType.DMA,
          pltpu.SemaphoreType.DMA,
          pltpu.SemaphoreType.DMA,
          pltpu.SemaphoreType.DMA,
          pltpu.VMEM_SHARED((NA, DIN), jnp.float32),
      ],
  )(table, src, dst, zeros_h)


def _sc_seg_rows_b(table, src, dst, zeros_h):
  def body(tab_h, src_h, dst_h, zeros_hh, out_h,
           is0, id0, r0, is1, id1, r1, smi0, smi1, smg0, smg1, acc):
    c = lax.axis_index("c")
    s = lax.axis_index("s")
    wid = c * NS + s
    ebase = wid * EPW
    _zero_acc(zeros_hh, acc, s)
    plsc.subcore_barrier()

    def idx_start(i, isb, idb, smi):
      b = ebase + i * CK
      pltpu.async_copy(src_h.at[pl.ds(b, CK)], isb, smi)
      pltpu.async_copy(dst_h.at[pl.ds(b, CK)], idb, smi)

    def idx_wait(i, isb, idb, smi):
      b = ebase + i * CK
      pltpu.make_async_copy(src_h.at[pl.ds(b, CK)], isb, smi).wait()
      pltpu.make_async_copy(dst_h.at[pl.ds(b, CK)], idb, smi).wait()

    pltpu.sync_copy(src_h.at[pl.ds(ebase, CK)], is0)
    pltpu.sync_copy(dst_h.at[pl.ds(ebase, CK)], id0)
    pltpu.async_copy(tab_h.at[is0], r0, smg0)
    idx_start(1, is1, id1, smi1)

    npair = NCH // 2

    def pair(j, carry):
      i0 = 2 * j
      idx_wait(i0 + 1, is1, id1, smi1)
      pltpu.async_copy(tab_h.at[is1], r1, smg1)
      pltpu.make_async_copy(tab_h.at[is0], r0, smg0).wait()
      pltpu.sync_copy(r0, acc.at[id0], add=True)

      @pl.when(j < npair - 1)
      def _():
        idx_start(i0 + 2, is0, id0, smi0)

      @pl.when(j < npair - 1)
      def _():
        idx_wait(i0 + 2, is0, id0, smi0)
        pltpu.async_copy(tab_h.at[is0], r0, smg0)

      pltpu.make_async_copy(tab_h.at[is1], r1, smg1).wait()
      pltpu.sync_copy(r1, acc.at[id1], add=True)

      @pl.when(j < npair - 1)
      def _():
        idx_start(i0 + 3, is1, id1, smi1)

      return carry

    lax.fori_loop(0, npair, pair, 0)
    plsc.subcore_barrier()
    _readout(acc, out_h, c, s)

  return pl.kernel(
      body,
      out_type=jax.ShapeDtypeStruct((NC * N, 2 * DIN), jnp.bfloat16),
      mesh=_sc_mesh(),
      compiler_params=pltpu.CompilerParams(use_tc_tiling_on_sc=False),
      scratch_types=[
          pltpu.VMEM((CK,), jnp.int32),
          pltpu.VMEM((CK,), jnp.int32),
          pltpu.VMEM((CK, 2 * DIN), jnp.bfloat16),
          pltpu.VMEM((CK,), jnp.int32),
          pltpu.VMEM((CK,), jnp.int32),
          pltpu.VMEM((CK, 2 * DIN), jnp.bfloat16),
          pltpu.SemaphoreType.DMA,
          pltpu.SemaphoreType.DMA,
          pltpu.SemaphoreType.DMA,
          pltpu.SemaphoreType.DMA,
          pltpu.VMEM_SHARED((NA, 2 * DIN), jnp.bfloat16),
      ],
  )(table, src, dst, zeros_h)


def _sc_gat_logits(als8, ald8, cvec, src, dst, zeros_h):
  def body(als_h, ald_h, c_h, src_h, dst_h, zeros_hh,
           ex_h, out_h, idx_s, idx_d, asv, adv, exb, cv, acc):
    c = lax.axis_index("c")
    s = lax.axis_index("s")
    wid = c * NS + s
    _zero_acc(zeros_hh, acc, s)
    pltpu.sync_copy(c_h, cv)
    plsc.subcore_barrier()

    def chunk(i, carry):
      base = wid * EPW + i * CK
      pltpu.sync_copy(src_h.at[pl.ds(base, CK)], idx_s)
      pltpu.sync_copy(dst_h.at[pl.ds(base, CK)], idx_d)
      pltpu.sync_copy(als_h.at[idx_s], asv)
      pltpu.sync_copy(ald_h.at[idx_d], adv)
      cvv = cv[...]

      def epair(p, carry2):
        j0 = 2 * p
        j1 = 2 * p + 1
        t0 = asv[j0, :] + adv[j0, :]
        t1 = asv[j1, :] + adv[j1, :]
        t0 = jnp.maximum(t0, 0.2 * t0) - cvv
        t1 = jnp.maximum(t1, 0.2 * t1) - cvv
        exb[j0, :] = jnp.exp(t0)
        exb[j1, :] = jnp.exp(t1)
        return carry2

      lax.fori_loop(0, CK // 2, epair, 0)
      pltpu.sync_copy(exb, ex_h.at[pl.ds(base, CK)])
      pltpu.sync_copy(exb, acc.at[idx_d], add=True)
      return carry

    lax.fori_loop(0, NCH, chunk, 0)
    plsc.subcore_barrier()
    _readout(acc, out_h, c, s)

  return pl.kernel(
      body,
      out_type=[
          jax.ShapeDtypeStruct((EP, 16), jnp.float32),
          jax.ShapeDtypeStruct((NC * N, 16), jnp.float32),
      ],
      mesh=_sc_mesh(),
      compiler_params=pltpu.CompilerParams(use_tc_tiling_on_sc=False),
      scratch_types=[
          pltpu.VMEM((CK,), jnp.int32),
          pltpu.VMEM((CK,), jnp.int32),
          pltpu.VMEM((CK, 16), jnp.float32),
          pltpu.VMEM((CK, 16), jnp.float32),
          pltpu.VMEM((CK, 16), jnp.float32),
          pltpu.VMEM((16,), jnp.float32),
          pltpu.VMEM_SHARED((NA, 16), jnp.float32),
      ],
  )(als8, ald8, cvec, src, dst, zeros_h)


def _sc_gat_agg(hh8, exw, r8, src, dst, zeros_h):
  def body(hh_h, ex_h, r_h, src_h, dst_h, zeros_hh, out_h,
           is0, id0, r0, e0, v0, is1, id1, r1, e1, v1,
           smi0, smi1, smg0, smg1, y, acc):
    c = lax.axis_index("c")
    s = lax.axis_index("s")
    wid = c * NS + s
    ebase = wid * EPW
    _zero_acc(zeros_hh, acc, s)
    plsc.subcore_barrier()

    def idx_start(i, isb, idb, smi):
      b = ebase + i * CKA
      pltpu.async_copy(src_h.at[pl.ds(b, CKA)], isb, smi)
      pltpu.async_copy(dst_h.at[pl.ds(b, CKA)], idb, smi)

    def idx_wait(i, isb, idb, smi):
      b = ebase + i * CKA
      pltpu.make_async_copy(src_h.at[pl.ds(b, CKA)], isb, smi).wait()
      pltpu.make_async_copy(dst_h.at[pl.ds(b, CKA)], idb, smi).wait()

    def fetch_start(i, isb, idb, rb, eb, vb, smg):
      b = ebase + i * CKA
      pltpu.async_copy(hh_h.at[isb], rb, smg)
      pltpu.async_copy(ex_h.at[pl.ds(b, CKA)], eb, smg)
      pltpu.async_copy(r_h.at[idb], vb, smg)

    def fetch_wait(i, isb, idb, rb, eb, vb, smg):
      b = ebase + i * CKA
      pltpu.make_async_copy(hh_h.at[isb], rb, smg).wait()
      pltpu.make_async_copy(ex_h.at[pl.ds(b, CKA)], eb, smg).wait()
      pltpu.make_async_copy(r_h.at[idb], vb, smg).wait()

    def compute_scatter(rb, eb, vb, idb):
      def epair(p, carry2):
        j0 = 2 * p
        j1 = 2 * p + 1
        aa = eb[j0, :] * vb[j0, :]
        ab = eb[j1, :] * vb[j1, :]

        def splat(av, h):
          return jnp.full((16,), av[h], jnp.float32).astype(jnp.bfloat16)

        w = [splat(aa, h) for h in range(4)] + [splat(ab, h) for h in range(4)]
        for cb in range(16):
          va = (w[0] * rb[j0, pl.ds(cb * 16, 16)]
                + w[1] * rb[j0, pl.ds(256 + cb * 16, 16)]
                + w[2] * rb[j0, pl.ds(512 + cb * 16, 16)]
                + w[3] * rb[j0, pl.ds(768 + cb * 16, 16)])
          vb2 = (w[4] * rb[j1, pl.ds(cb * 16, 16)]
                 + w[5] * rb[j1, pl.ds(256 + cb * 16, 16)]
                 + w[6] * rb[j1, pl.ds(512 + cb * 16, 16)]
                 + w[7] * rb[j1, pl.ds(768 + cb * 16, 16)])
          y[j0, pl.ds(cb * 16, 16)] = va
          y[j1, pl.ds(cb * 16, 16)] = vb2
        return carry2

      lax.fori_loop(0, CKA // 2, epair, 0)
      pltpu.sync_copy(y, acc.at[idb], add=True)

    pltpu.sync_copy(src_h.at[pl.ds(ebase, CKA)], is0)
    pltpu.sync_copy(dst_h.at[pl.ds(ebase, CKA)], id0)
    fetch_start(0, is0, id0, r0, e0, v0, smg0)
    idx_start(1, is1, id1, smi1)

    npair = NCHA // 2

    def pair(j, carry):
      i0 = 2 * j
      idx_wait(i0 + 1, is1, id1, smi1)
      fetch_start(i0 + 1, is1, id1, r1, e1, v1, smg1)
      fetch_wait(i0, is0, id0, r0, e0, v0, smg0)
      compute_scatter(r0, e0, v0, id0)

      @pl.when(j < npair - 1)
      def _():
        idx_start(i0 + 2, is0, id0, smi0)
        idx_wait(i0 + 2, is0, id0, smi0)
        fetch_start(i0 + 2, is0, id0, r0, e0, v0, smg0)

      fetch_wait(i0 + 1, is1, id1, r1, e1, v1, smg1)
      compute_scatter(r1, e1, v1, id1)

      @pl.when(j < npair - 1)
      def _():
        idx_start(i0 + 3, is1, id1, smi1)

      return carry

    lax.fori_loop(0, npair, pair, 0)
    plsc.subcore_barrier()
    _readout(acc, out_h, c, s)

  bufset = [
      pltpu.VMEM((CKA,), jnp.int32),
      pltpu.VMEM((CKA,), jnp.int32),
      pltpu.VMEM((CKA, 8 * DIN), jnp.bfloat16),
      pltpu.VMEM((CKA, 16), jnp.float32),
      pltpu.VMEM((CKA, 16), jnp.float32),
  ]
  return pl.kernel(
      body,
      out_type=jax.ShapeDtypeStruct((NC * N, 2 * DIN), jnp.bfloat16),
      mesh=_sc_mesh(),
      compiler_params=pltpu.CompilerParams(use_tc_tiling_on_sc=False),
      scratch_types=bufset + bufset + [
          pltpu.SemaphoreType.DMA,
          pltpu.SemaphoreType.DMA,
          pltpu.SemaphoreType.DMA,
          pltpu.SemaphoreType.DMA,
          pltpu.VMEM((CKA, 2 * DIN), jnp.bfloat16),
          pltpu.VMEM_SHARED((NA, 2 * DIN), jnp.bfloat16),
      ],
  )(hh8, exw, r8, src, dst, zeros_h)


def _relu(v):
  return jnp.maximum(v, 0.0)


def _k1(degp0, degp1, x):
  def body(d0, d1, xr, xp, dv):
    deg = d0[:, :1] + d1[:, :1] + 1.0
    dinv = lax.rsqrt(deg)
    dvb = jnp.broadcast_to(dinv, (BN, DIN))
    dv[...] = dvb
    xp[...] = xr[...] * dvb

  bs16 = pl.BlockSpec((BN, 16), lambda i: (i, 0))
  bs128 = pl.BlockSpec((BN, DIN), lambda i: (i, 0))
  return pl.pallas_call(
      body,
      grid=(NB,),
      in_specs=[bs16, bs16, bs128],
      out_specs=[bs128, bs128],
      out_shape=[
          jax.ShapeDtypeStruct((N, DIN), jnp.float32),
          jax.ShapeDtypeStruct((N, DIN), jnp.float32),
      ],
  )(degp0, degp1, x)


def _k2(a0, a1, xp, dv, W1, b1):
  def body(a0r, a1r, xpr, dvr, w, b, lo, hi, h1pb):
    pre = dvr[...] * (a0r[...] + a1r[...] + xpr[...])
    h = _relu(jnp.dot(pre, w[...], preferred_element_type=jnp.float32) + b[...])
    h1p = h * dvr[:, :1]
    lo[...] = h1p[:, :DIN]
    hi[...] = h1p[:, DIN:]
    h1pb[...] = h1p.astype(jnp.bfloat16)

  bs128 = pl.BlockSpec((BN, DIN), lambda i: (i, 0))
  return pl.pallas_call(
      body,
      grid=(NB,),
      in_specs=[bs128, bs128, bs128, bs128,
                pl.BlockSpec((DIN, DH), lambda i: (0, 0)),
                pl.BlockSpec((1, DH), lambda i: (0, 0))],
      out_specs=[bs128, bs128, pl.BlockSpec((BN, DH), lambda i: (i, 0))],
      out_shape=[
          jax.ShapeDtypeStruct((N, DIN), jnp.float32),
          jax.ShapeDtypeStruct((N, DIN), jnp.float32),
          jax.ShapeDtypeStruct((N, DH), jnp.bfloat16),
      ],
  )(a0, a1, xp, dv, W1, b1)


def _k3(ag0f, ag1f, lo, hi, dv, W2, b2, W3, As, Ad):
  def body(ag0, ag1, lor, hir, dvr, w2, b2r, w3, asr, adr,
           hh8lo, hh8hi, hhb, als, ald):
    h1p = jnp.concatenate([lor[...], hir[...]], axis=1)
    agg = ag0[...].astype(jnp.float32) + ag1[...].astype(jnp.float32)
    pre = dvr[:, :1] * (agg + h1p)
    h2 = _relu(jnp.dot(pre, w2[...], preferred_element_type=jnp.float32)
               + b2r[...])
    hh = jnp.dot(h2, w3[...], preferred_element_type=jnp.float32)
    als[...] = jnp.dot(hh, asr[...], preferred_element_type=jnp.float32)
    ald[...] = jnp.dot(hh, adr[...], preferred_element_type=jnp.float32)
    hh8lo[...] = jnp.concatenate(
        [hh[:, 0:128], hh[:, 256:384], hh[:, 512:640], hh[:, 768:896]], axis=1)
    hh8hi[...] = jnp.concatenate(
        [hh[:, 128:256], hh[:, 384:512], hh[:, 640:768], hh[:, 896:1024]],
        axis=1)
    hhb[...] = hh.astype(jnp.bfloat16)

  bs128 = pl.BlockSpec((BN, DIN), lambda i: (i, 0))
  bs256b = pl.BlockSpec((BN, 2 * DIN), lambda i: (i, 0))
  bs512 = pl.BlockSpec((BN, 4 * DIN), lambda i: (i, 0))
  bs1024 = pl.BlockSpec((BN, 8 * DIN), lambda i: (i, 0))
  bs16 = pl.BlockSpec((BN, 16), lambda i: (i, 0))
  return pl.pallas_call(
      body,
      grid=(NB,),
      in_specs=[bs256b, bs256b, bs128, bs128, bs128,
                pl.BlockSpec((DH, DH), lambda i: (0, 0)),
                pl.BlockSpec((1, DH), lambda i: (0, 0)),
                pl.BlockSpec((DH, H * DH), lambda i: (0, 0)),
                pl.BlockSpec((H * DH, 16), lambda i: (0, 0)),
                pl.BlockSpec((H * DH, 16), lambda i: (0, 0))],
      out_specs=[bs512, bs512, bs1024, bs16, bs16],
      out_shape=[
          jax.ShapeDtypeStruct((N, 4 * DIN), jnp.float32),
          jax.ShapeDtypeStruct((N, 4 * DIN), jnp.float32),
          jax.ShapeDtypeStruct((N, 8 * DIN), jnp.bfloat16),
          jax.ShapeDtypeStruct((N, 16), jnp.float32),
          jax.ShapeDtypeStruct((N, 16), jnp.float32),
      ],
  )(ag0f, ag1f, lo, hi, dv, W2, b2, W3, As, Ad)


def _k3b(als, ald):
  def body(alsr, aldr, ms, md, cout):
    i = pl.program_id(0)

    @pl.when(i == 0)
    def _():
      ms[...] = jnp.full((1, 16), -1e30, jnp.float32)
      md[...] = jnp.full((1, 16), -1e30, jnp.float32)

    ms[...] = jnp.maximum(ms[...], jnp.max(alsr[...], axis=0, keepdims=True))
    md[...] = jnp.maximum(md[...], jnp.max(aldr[...], axis=0, keepdims=True))

    @pl.when(i == NB - 1)
    def _():
      a = ms[...]
      b = md[...]
      cout[...] = jnp.maximum(a, 0.2 * a) + jnp.maximum(b, 0.2 * b)

  bs16 = pl.BlockSpec((BN, 16), lambda i: (i, 0))
  os = pl.BlockSpec((1, 16), lambda i: (0, 0))
  return pl.pallas_call(
      body,
      grid=(NB,),
      in_specs=[bs16, bs16],
      out_specs=[os, os, os],
      out_shape=[jax.ShapeDtypeStruct((1, 16), jnp.float32)] * 3,
  )(als, ald)


def _k4(s0, s1, als, ald, C):
  def body(s0r, s1r, alsr, aldr, cr, r8, ws8):
    t = alsr[...] + aldr[...]
    t = jnp.maximum(t, 0.2 * t) - cr[...]
    exs = jnp.exp(t)
    stot = s0r[...] + s1r[...] + exs
    lane = lax.broadcasted_iota(jnp.int32, (BN, 16), 1)
    r = jnp.where(lane < H, 1.0 / (stot + 1e-16), 0.0)
    r8[...] = r
    ws8[...] = exs * r

  bs16 = pl.BlockSpec((BN, 16), lambda i: (i, 0))
  return pl.pallas_call(
      body,
      grid=(NB,),
      in_specs=[bs16, bs16, bs16, bs16,
                pl.BlockSpec((1, 16), lambda i: (0, 0))],
      out_specs=[bs16, bs16],
      out_shape=[jax.ShapeDtypeStruct((N, 16), jnp.float32)] * 2,
  )(s0, s1, als, ald, C)


def _k5(ag0, ag1, hlo, hhi, r8, ws8, b3f, batchT):
  def body(a0r, a1r, hlor, hhir, r8r, ws8r, b3r, btr, pool, cnt):
    i = pl.program_id(0)
    agg = a0r[...].astype(jnp.float32) + a1r[...].astype(jnp.float32)
    ws = ws8r[...]
    hlo_v = hlor[...]
    hhi_v = hhir[...]
    self_lo = ws[:, 0:1] * hlo_v[:, 0:128]
    self_hi = ws[:, 0:1] * hhi_v[:, 0:128]
    for h in range(1, H):
      self_lo = self_lo + ws[:, h:h + 1] * hlo_v[:, h * 128:(h + 1) * 128]
      self_hi = self_hi + ws[:, h:h + 1] * hhi_v[:, h * 128:(h + 1) * 128]
    m = 0.25 * (agg + jnp.concatenate([self_lo, self_hi], axis=1))
    h3 = _relu(m + b3r[...])
    mask = (lax.broadcasted_iota(jnp.int32, (BN, G), 1) == btr[...])
    mf = jnp.where(mask, 1.0, 0.0)

    @pl.when(i == 0)
    def _():
      pool[...] = jnp.zeros((G, DH), jnp.float32)
      cnt[...] = jnp.zeros((G, 8), jnp.float32)

    dn = (((0,), (0,)), ((), ()))
    pool[...] += lax.dot_general(mf, h3, dn,
                                 preferred_element_type=jnp.float32)
    cnt[...] += lax.dot_general(mf, jnp.ones((BN, 8), jnp.float32), dn,
                                preferred_element_type=jnp.float32)

  bs256b = pl.BlockSpec((BN, 2 * DIN), lambda i: (i, 0))
  bs512 = pl.BlockSpec((BN, 4 * DIN), lambda i: (i, 0))
  bs16 = pl.BlockSpec((BN, 16), lambda i: (i, 0))
  return pl.pallas_call(
      body,
      grid=(NB,),
      in_specs=[bs256b, bs256b, bs512, bs512, bs16, bs16,
                pl.BlockSpec((1, DH), lambda i: (0, 0)),
                pl.BlockSpec((BN, 1), lambda i: (i, 0))],
      out_specs=[pl.BlockSpec((G, DH), lambda i: (0, 0)),
                 pl.BlockSpec((G, 8), lambda i: (0, 0))],
      out_shape=[
          jax.ShapeDtypeStruct((G, DH), jnp.float32),
          jax.ShapeDtypeStruct((G, 8), jnp.float32),
      ],
  )(ag0, ag1, hlo, hhi, r8, ws8, b3f, batchT)


def _k6(pool, cnt, W4, b4):
  def body(poolr, cntr, w4, b4r, out):
    gr = poolr[...] / jnp.maximum(cntr[:, :1], 1.0)
    out[...] = jnp.dot(gr, w4[...], preferred_element_type=jnp.float32) + b4r[...]

  return pl.pallas_call(
      body,
      in_specs=[pl.BlockSpec((G, DH), lambda: (0, 0)),
                pl.BlockSpec((G, 8), lambda: (0, 0)),
                pl.BlockSpec((DH, 4), lambda: (0, 0)),
                pl.BlockSpec((1, 4), lambda: (0, 0))],
      out_specs=pl.BlockSpec((G, 4), lambda: (0, 0)),
      out_shape=jax.ShapeDtypeStruct((G, 4), jnp.float32),
  )(pool, cnt, W4, b4)


def kernel(x, edge_index, batch, W1, b1, W2, b2, W3, a_src, a_dst, b3,
           We, be, Wm, bm, Wb, bb, Wp, bp):
  npad = EP - E
  src = jnp.concatenate([edge_index[0], jnp.zeros((npad,), jnp.int32)])
  dst = jnp.concatenate([edge_index[1], jnp.full((npad,), N, jnp.int32)])

  zeros16 = jnp.zeros((NPS, 16), jnp.float32)
  zeros128 = jnp.zeros((NPS, DIN), jnp.float32)
  ones_ck = jnp.ones((CKC, 16), jnp.float32)

  As = jnp.zeros((H * DH, 16), jnp.float32)
  Ad = jnp.zeros((H * DH, 16), jnp.float32)
  for h in range(H):
    As = As.at[h * DH:(h + 1) * DH, h].set(a_src[h])
    Ad = Ad.at[h * DH:(h + 1) * DH, h].set(a_dst[h])

  W4 = jnp.concatenate([We, Wm, Wb, Wp], axis=1)
  b4 = jnp.concatenate([be, bm, bb, bp]).reshape(1, 4)

  degp = _sc_count(dst, ones_ck, zeros16)
  xp, dv = _k1(degp[:N], degp[N:], x)
  a1p = _sc_seg_rows(xp, src, dst, zeros128)
  h1plo, h1phi, h1pb = _k2(a1p[:N], a1p[N:], xp, dv, W1, b1.reshape(1, DH))

  zeros256b = jnp.zeros((NPS, 2 * DIN), jnp.bfloat16)
  a2 = _sc_seg_rows_b(h1pb, src, dst, zeros256b)
  hh8lo, hh8hi, hhb, als, ald = _k3(
      a2[:N], a2[N:], h1plo, h1phi, dv, W2, b2.reshape(1, DH), W3, As, Ad)

  _, _, C = _k3b(als, ald)
  pad8 = jnp.zeros((NA - N, 16), jnp.float32)
  ex, sp = _sc_gat_logits(jnp.concatenate([als, pad8]),
                          jnp.concatenate([ald, pad8]),
                          C.reshape(16), src, dst, zeros16)
  r8, ws8 = _k4(sp[:N], sp[N:], als, ald, C)
  r8p = jnp.concatenate([r8, pad8])
  a3 = _sc_gat_agg(hhb, ex, r8p, src, dst, zeros256b)

  pool, cnt = _k5(a3[:N], a3[N:], hh8lo, hh8hi,
                  r8, ws8, b3.reshape(1, DH), batch.reshape(N, 1))
  return _k6(pool, cnt, W4, b4)


# speedup vs baseline: 1.4029x; 1.0424x over previous
import functools

import jax
import jax.numpy as jnp
from jax import lax
from jax.experimental import pallas as pl
from jax.experimental.pallas import tpu as pltpu
from jax.experimental.pallas import tpu_sc as plsc

N = 10000
E = 320000
DIN = 128
DH = 256
H = 4
G = 64

NC = 2
NS = 16
NW = NC * NS
EP = 327680
EPW = EP // NW
CK = 128
NCH = EPW // CK
CKC = 512
NCHC = EPW // CKC
CKL = 512
NCHL = EPW // CKL
CKA = 32
NCHA = EPW // CKA
NA = N + 8
NPS = 624
TBASE = NS * NPS
ZTAIL = NA - TBASE
RTAIL = N - TBASE

BN = 400
NB = N // BN


def _sc_mesh():
  return plsc.VectorSubcoreMesh(core_axis_name="c", subcore_axis_name="s",
                                num_cores=NC, num_subcores=NS)


def _wid():
  return lax.axis_index("c") * NS + lax.axis_index("s")


def _zero_acc(zeros_hh, acc, s):
  pltpu.sync_copy(zeros_hh, acc.at[pl.ds(s * NPS, NPS)])

  @pl.when(s == 0)
  def _():
    pltpu.sync_copy(zeros_hh.at[pl.ds(0, ZTAIL)], acc.at[pl.ds(TBASE, ZTAIL)])


def _readout(acc, out_h, c, s):
  pltpu.sync_copy(acc.at[pl.ds(s * NPS, NPS)],
                  out_h.at[pl.ds(c * N + s * NPS, NPS)])

  @pl.when(s == 0)
  def _():
    pltpu.sync_copy(acc.at[pl.ds(TBASE, RTAIL)],
                    out_h.at[pl.ds(c * N + TBASE, RTAIL)])


def _sc_count(dst, ones_h, zeros_h):
  def body(dst_h, ones_hh, zeros_hh, out_h, id0, id1, ones_v, smi0, smi1, acc):
    c = lax.axis_index("c")
    s = lax.axis_index("s")
    wid = c * NS + s
    ebase = wid * EPW
    _zero_acc(zeros_hh, acc, s)
    pltpu.sync_copy(ones_hh, ones_v)
    plsc.subcore_barrier()

    def idx_start(i, idb, smi):
      pltpu.async_copy(dst_h.at[pl.ds(ebase + i * CKC, CKC)], idb, smi)

    def idx_wait(i, idb, smi):
      pltpu.make_async_copy(dst_h.at[pl.ds(ebase + i * CKC, CKC)],
                            idb, smi).wait()

    idx_start(0, id0, smi0)
    idx_start(1, id1, smi1)
    npair = NCHC // 2

    def pair(j, carry):
      i0 = 2 * j
      idx_wait(i0, id0, smi0)
      pltpu.sync_copy(ones_v, acc.at[id0], add=True)

      @pl.when(j < npair - 1)
      def _():
        idx_start(i0 + 2, id0, smi0)

      idx_wait(i0 + 1, id1, smi1)
      pltpu.sync_copy(ones_v, acc.at[id1], add=True)

      @pl.when(j < npair - 1)
      def _():
        idx_start(i0 + 3, id1, smi1)

      return carry

    lax.fori_loop(0, npair, pair, 0)
    plsc.subcore_barrier()
    _readout(acc, out_h, c, s)

  return pl.kernel(
      body,
      out_type=jax.ShapeDtypeStruct((NC * N, 16), jnp.float32),
      mesh=_sc_mesh(),
      compiler_params=pltpu.CompilerParams(use_tc_tiling_on_sc=False),
      scratch_types=[
          pltpu.VMEM((CKC,), jnp.int32),
          pltpu.VMEM((CKC,), jnp.int32),
          pltpu.VMEM((CKC, 16), jnp.float32),
          pltpu.SemaphoreType.DMA,
          pltpu.SemaphoreType.DMA,
          pltpu.VMEM_SHARED((NA, 16), jnp.float32),
      ],
  )(dst, ones_h, zeros_h)


def _sc_seg_rows(table, src, dst, zeros_h):
  def body(tab_h, src_h, dst_h, zeros_hh, out_h,
           is0, id0, r0, is1, id1, r1, smi0, smi1, smg0, smg1, acc):
    c = lax.axis_index("c")
    s = lax.axis_index("s")
    wid = c * NS + s
    ebase = wid * EPW
    _zero_acc(zeros_hh, acc, s)
    plsc.subcore_barrier()

    def idx_start(i, isb, idb, smi):
      b = ebase + i * CK
      pltpu.async_copy(src_h.at[pl.ds(b, CK)], isb, smi)
      pltpu.async_copy(dst_h.at[pl.ds(b, CK)], idb, smi)

    def idx_wait(i, isb, idb, smi):
      b = ebase + i * CK
      pltpu.make_async_copy(src_h.at[pl.ds(b, CK)], isb, smi).wait()
      pltpu.make_async_copy(dst_h.at[pl.ds(b, CK)], idb, smi).wait()

    pltpu.sync_copy(src_h.at[pl.ds(ebase, CK)], is0)
    pltpu.sync_copy(dst_h.at[pl.ds(ebase, CK)], id0)
    pltpu.async_copy(tab_h.at[is0], r0, smg0)
    idx_start(1, is1, id1, smi1)

    npair = NCH // 2

    def pair(j, carry):
      i0 = 2 * j
      idx_wait(i0 + 1, is1, id1, smi1)
      pltpu.async_copy(tab_h.at[is1], r1, smg1)
      pltpu.make_async_copy(tab_h.at[is0], r0, smg0).wait()
      pltpu.sync_copy(r0, acc.at[id0], add=True)

      @pl.when(j < npair - 1)
      def _():
        idx_start(i0 + 2, is0, id0, smi0)

      @pl.when(j < npair - 1)
      def _():
        idx_wait(i0 + 2, is0, id0, smi0)
        pltpu.async_copy(tab_h.at[is0], r0, smg0)

      pltpu.make_async_copy(tab_h.at[is1], r1, smg1).wait()
      pltpu.sync_copy(r1, acc.at[id1], add=True)

      @pl.when(j < npair - 1)
      def _():
        idx_start(i0 + 3, is1, id1, smi1)

      return carry

    lax.fori_loop(0, npair, pair, 0)
    plsc.subcore_barrier()
    _readout(acc, out_h, c, s)

  return pl.kernel(
      body,
      out_type=jax.ShapeDtypeStruct((NC * N, DIN), jnp.float32),
      mesh=_sc_mesh(),
      compiler_params=pltpu.CompilerParams(use_tc_tiling_on_sc=False),
      scratch_types=[
          pltpu.VMEM((CK,), jnp.int32),
          pltpu.VMEM((CK,), jnp.int32),
          pltpu.VMEM((CK, DIN), jnp.float32),
          pltpu.VMEM((CK,), jnp.int32),
          pltpu.VMEM((CK,), jnp.int32),
          pltpu.VMEM((CK, DIN), jnp.float32),
          pltpu.SemaphoreType.DMA,
          pltpu.SemaphoreType.DMA,
          pltpu.SemaphoreType.DMA,
          pltpu.SemaphoreType.DMA,
          pltpu.VMEM_SHARED((NA, DIN), jnp.float32),
      ],
  )(table, src, dst, zeros_h)


def _sc_seg_rows_b(table, src, dst, zeros_h):
  def body(tab_h, src_h, dst_h, zeros_hh, out_h,
           is0, id0, r0, is1, id1, r1, smi0, smi1, smg0, smg1, acc):
    c = lax.axis_index("c")
    s = lax.axis_index("s")
    wid = c * NS + s
    ebase = wid * EPW
    _zero_acc(zeros_hh, acc, s)
    plsc.subcore_barrier()

    def idx_start(i, isb, idb, smi):
      b = ebase + i * CK
      pltpu.async_copy(src_h.at[pl.ds(b, CK)], isb, smi)
      pltpu.async_copy(dst_h.at[pl.ds(b, CK)], idb, smi)

    def idx_wait(i, isb, idb, smi):
      b = ebase + i * CK
      pltpu.make_async_copy(src_h.at[pl.ds(b, CK)], isb, smi).wait()
      pltpu.make_async_copy(dst_h.at[pl.ds(b, CK)], idb, smi).wait()

    pltpu.sync_copy(src_h.at[pl.ds(ebase, CK)], is0)
    pltpu.sync_copy(dst_h.at[pl.ds(ebase, CK)], id0)
    pltpu.async_copy(tab_h.at[is0], r0, smg0)
    idx_start(1, is1, id1, smi1)

    npair = NCH // 2

    def pair(j, carry):
      i0 = 2 * j
      idx_wait(i0 + 1, is1, id1, smi1)
      pltpu.async_copy(tab_h.at[is1], r1, smg1)
      pltpu.make_async_copy(tab_h.at[is0], r0, smg0).wait()
      pltpu.sync_copy(r0, acc.at[id0], add=True)

      @pl.when(j < npair - 1)
      def _():
        idx_start(i0 + 2, is0, id0, smi0)

      @pl.when(j < npair - 1)
      def _():
        idx_wait(i0 + 2, is0, id0, smi0)
        pltpu.async_copy(tab_h.at[is0], r0, smg0)

      pltpu.make_async_copy(tab_h.at[is1], r1, smg1).wait()
      pltpu.sync_copy(r1, acc.at[id1], add=True)

      @pl.when(j < npair - 1)
      def _():
        idx_start(i0 + 3, is1, id1, smi1)

      return carry

    lax.fori_loop(0, npair, pair, 0)
    plsc.subcore_barrier()
    _readout(acc, out_h, c, s)

  return pl.kernel(
      body,
      out_type=jax.ShapeDtypeStruct((NC * N, 2 * DIN), jnp.bfloat16),
      mesh=_sc_mesh(),
      compiler_params=pltpu.CompilerParams(use_tc_tiling_on_sc=False),
      scratch_types=[
          pltpu.VMEM((CK,), jnp.int32),
          pltpu.VMEM((CK,), jnp.int32),
          pltpu.VMEM((CK, 2 * DIN), jnp.bfloat16),
          pltpu.VMEM((CK,), jnp.int32),
          pltpu.VMEM((CK,), jnp.int32),
          pltpu.VMEM((CK, 2 * DIN), jnp.bfloat16),
          pltpu.SemaphoreType.DMA,
          pltpu.SemaphoreType.DMA,
          pltpu.SemaphoreType.DMA,
          pltpu.SemaphoreType.DMA,
          pltpu.VMEM_SHARED((NA, 2 * DIN), jnp.bfloat16),
      ],
  )(table, src, dst, zeros_h)


def _sc_gat_logits(als8, ald8, cvec, src, dst, zeros_h):
  def body(als_h, ald_h, c_h, src_h, dst_h, zeros_hh,
           ex_h, out_h, idx_s, idx_d, asv, adv, exb, cv, acc):
    c = lax.axis_index("c")
    s = lax.axis_index("s")
    wid = c * NS + s
    _zero_acc(zeros_hh, acc, s)
    pltpu.sync_copy(c_h, cv)
    plsc.subcore_barrier()

    def chunk(i, carry):
      base = wid * EPW + i * CKL
      pltpu.sync_copy(src_h.at[pl.ds(base, CKL)], idx_s)
      pltpu.sync_copy(dst_h.at[pl.ds(base, CKL)], idx_d)
      pltpu.sync_copy(als_h.at[idx_s], asv)
      pltpu.sync_copy(ald_h.at[idx_d], adv)
      cvv = cv[...]

      def epair(p, carry2):
        j0 = 2 * p
        j1 = 2 * p + 1
        t0 = asv[j0, :] + adv[j0, :]
        t1 = asv[j1, :] + adv[j1, :]
        t0 = jnp.maximum(t0, 0.2 * t0) - cvv
        t1 = jnp.maximum(t1, 0.2 * t1) - cvv
        exb[j0, :] = jnp.exp(t0)
        exb[j1, :] = jnp.exp(t1)
        return carry2

      lax.fori_loop(0, CKL // 2, epair, 0)
      pltpu.sync_copy(exb, ex_h.at[pl.ds(base, CKL)])
      pltpu.sync_copy(exb, acc.at[idx_d], add=True)
      return carry

    lax.fori_loop(0, NCHL, chunk, 0)
    plsc.subcore_barrier()
    _readout(acc, out_h, c, s)

  return pl.kernel(
      body,
      out_type=[
          jax.ShapeDtypeStruct((EP, 16), jnp.float32),
          jax.ShapeDtypeStruct((NC * N, 16), jnp.float32),
      ],
      mesh=_sc_mesh(),
      compiler_params=pltpu.CompilerParams(use_tc_tiling_on_sc=False),
      scratch_types=[
          pltpu.VMEM((CKL,), jnp.int32),
          pltpu.VMEM((CKL,), jnp.int32),
          pltpu.VMEM((CKL, 16), jnp.float32),
          pltpu.VMEM((CKL, 16), jnp.float32),
          pltpu.VMEM((CKL, 16), jnp.float32),
          pltpu.VMEM((16,), jnp.float32),
          pltpu.VMEM_SHARED((NA, 16), jnp.float32),
      ],
  )(als8, ald8, cvec, src, dst, zeros_h)


def _sc_gat_agg(hh8, exw, r8, src, dst, zeros_h):
  def body(hh_h, ex_h, r_h, src_h, dst_h, zeros_hh, out_h,
           is0, id0, r0, e0, v0, is1, id1, r1, e1, v1,
           smi0, smi1, smg0, smg1, y, acc):
    c = lax.axis_index("c")
    s = lax.axis_index("s")
    wid = c * NS + s
    ebase = wid * EPW
    _zero_acc(zeros_hh, acc, s)
    plsc.subcore_barrier()

    def idx_start(i, isb, idb, smi):
      b = ebase + i * CKA
      pltpu.async_copy(src_h.at[pl.ds(b, CKA)], isb, smi)
      pltpu.async_copy(dst_h.at[pl.ds(b, CKA)], idb, smi)

    def idx_wait(i, isb, idb, smi):
      b = ebase + i * CKA
      pltpu.make_async_copy(src_h.at[pl.ds(b, CKA)], isb, smi).wait()
      pltpu.make_async_copy(dst_h.at[pl.ds(b, CKA)], idb, smi).wait()

    def fetch_start(i, isb, idb, rb, eb, vb, smg):
      b = ebase + i * CKA
      pltpu.async_copy(hh_h.at[isb], rb, smg)
      pltpu.async_copy(ex_h.at[pl.ds(b, CKA)], eb, smg)
      pltpu.async_copy(r_h.at[idb], vb, smg)

    def fetch_wait(i, isb, idb, rb, eb, vb, smg):
      b = ebase + i * CKA
      pltpu.make_async_copy(hh_h.at[isb], rb, smg).wait()
      pltpu.make_async_copy(ex_h.at[pl.ds(b, CKA)], eb, smg).wait()
      pltpu.make_async_copy(r_h.at[idb], vb, smg).wait()

    def compute_scatter(rb, eb, vb, idb):
      def epair(p, carry2):
        j0 = 2 * p
        j1 = 2 * p + 1
        aa = eb[j0, :] * vb[j0, :]
        ab = eb[j1, :] * vb[j1, :]

        def splat(av, h):
          return jnp.full((16,), av[h], jnp.float32).astype(jnp.bfloat16)

        w = [splat(aa, h) for h in range(4)] + [splat(ab, h) for h in range(4)]
        for cb in range(16):
          va = (w[0] * rb[j0, pl.ds(cb * 16, 16)]
                + w[1] * rb[j0, pl.ds(256 + cb * 16, 16)]
                + w[2] * rb[j0, pl.ds(512 + cb * 16, 16)]
                + w[3] * rb[j0, pl.ds(768 + cb * 16, 16)])
          vb2 = (w[4] * rb[j1, pl.ds(cb * 16, 16)]
                 + w[5] * rb[j1, pl.ds(256 + cb * 16, 16)]
                 + w[6] * rb[j1, pl.ds(512 + cb * 16, 16)]
                 + w[7] * rb[j1, pl.ds(768 + cb * 16, 16)])
          y[j0, pl.ds(cb * 16, 16)] = va
          y[j1, pl.ds(cb * 16, 16)] = vb2
        return carry2

      lax.fori_loop(0, CKA // 2, epair, 0)
      pltpu.sync_copy(y, acc.at[idb], add=True)

    pltpu.sync_copy(src_h.at[pl.ds(ebase, CKA)], is0)
    pltpu.sync_copy(dst_h.at[pl.ds(ebase, CKA)], id0)
    fetch_start(0, is0, id0, r0, e0, v0, smg0)
    idx_start(1, is1, id1, smi1)

    npair = NCHA // 2

    def pair(j, carry):
      i0 = 2 * j
      idx_wait(i0 + 1, is1, id1, smi1)
      fetch_start(i0 + 1, is1, id1, r1, e1, v1, smg1)
      fetch_wait(i0, is0, id0, r0, e0, v0, smg0)
      compute_scatter(r0, e0, v0, id0)

      @pl.when(j < npair - 1)
      def _():
        idx_start(i0 + 2, is0, id0, smi0)
        idx_wait(i0 + 2, is0, id0, smi0)
        fetch_start(i0 + 2, is0, id0, r0, e0, v0, smg0)

      fetch_wait(i0 + 1, is1, id1, r1, e1, v1, smg1)
      compute_scatter(r1, e1, v1, id1)

      @pl.when(j < npair - 1)
      def _():
        idx_start(i0 + 3, is1, id1, smi1)

      return carry

    lax.fori_loop(0, npair, pair, 0)
    plsc.subcore_barrier()
    _readout(acc, out_h, c, s)

  bufset = [
      pltpu.VMEM((CKA,), jnp.int32),
      pltpu.VMEM((CKA,), jnp.int32),
      pltpu.VMEM((CKA, 8 * DIN), jnp.bfloat16),
      pltpu.VMEM((CKA, 16), jnp.float32),
      pltpu.VMEM((CKA, 16), jnp.float32),
  ]
  return pl.kernel(
      body,
      out_type=jax.ShapeDtypeStruct((NC * N, 2 * DIN), jnp.bfloat16),
      mesh=_sc_mesh(),
      compiler_params=pltpu.CompilerParams(use_tc_tiling_on_sc=False),
      scratch_types=bufset + bufset + [
          pltpu.SemaphoreType.DMA,
          pltpu.SemaphoreType.DMA,
          pltpu.SemaphoreType.DMA,
          pltpu.SemaphoreType.DMA,
          pltpu.VMEM((CKA, 2 * DIN), jnp.bfloat16),
          pltpu.VMEM_SHARED((NA, 2 * DIN), jnp.bfloat16),
      ],
  )(hh8, exw, r8, src, dst, zeros_h)


def _relu(v):
  return jnp.maximum(v, 0.0)


def _k1(degp0, degp1, x):
  def body(d0, d1, xr, xp, dv):
    deg = d0[:, :1] + d1[:, :1] + 1.0
    dinv = lax.rsqrt(deg)
    dvb = jnp.broadcast_to(dinv, (BN, DIN))
    dv[...] = dvb
    xp[...] = xr[...] * dvb

  bs16 = pl.BlockSpec((BN, 16), lambda i: (i, 0))
  bs128 = pl.BlockSpec((BN, DIN), lambda i: (i, 0))
  return pl.pallas_call(
      body,
      grid=(NB,),
      in_specs=[bs16, bs16, bs128],
      out_specs=[bs128, bs128],
      out_shape=[
          jax.ShapeDtypeStruct((N, DIN), jnp.float32),
          jax.ShapeDtypeStruct((N, DIN), jnp.float32),
      ],
  )(degp0, degp1, x)


def _k2(a0, a1, xp, dv, W1, b1):
  def body(a0r, a1r, xpr, dvr, w, b, lo, hi, h1pb):
    pre = dvr[...] * (a0r[...] + a1r[...] + xpr[...])
    h = _relu(jnp.dot(pre, w[...], preferred_element_type=jnp.float32) + b[...])
    h1p = h * dvr[:, :1]
    lo[...] = h1p[:, :DIN]
    hi[...] = h1p[:, DIN:]
    h1pb[...] = h1p.astype(jnp.bfloat16)

  bs128 = pl.BlockSpec((BN, DIN), lambda i: (i, 0))
  return pl.pallas_call(
      body,
      grid=(NB,),
      in_specs=[bs128, bs128, bs128, bs128,
                pl.BlockSpec((DIN, DH), lambda i: (0, 0)),
                pl.BlockSpec((1, DH), lambda i: (0, 0))],
      out_specs=[bs128, bs128, pl.BlockSpec((BN, DH), lambda i: (i, 0))],
      out_shape=[
          jax.ShapeDtypeStruct((N, DIN), jnp.float32),
          jax.ShapeDtypeStruct((N, DIN), jnp.float32),
          jax.ShapeDtypeStruct((N, DH), jnp.bfloat16),
      ],
  )(a0, a1, xp, dv, W1, b1)


def _k3(ag0f, ag1f, lo, hi, dv, W2, b2, W3, As, Ad):
  def body(ag0, ag1, lor, hir, dvr, w2, b2r, w3, asr, adr,
           hh8lo, hh8hi, hhb, als, ald):
    h1p = jnp.concatenate([lor[...], hir[...]], axis=1)
    agg = ag0[...].astype(jnp.float32) + ag1[...].astype(jnp.float32)
    pre = dvr[:, :1] * (agg + h1p)
    h2 = _relu(jnp.dot(pre, w2[...], preferred_element_type=jnp.float32)
               + b2r[...])
    hh = jnp.dot(h2, w3[...], preferred_element_type=jnp.float32)
    als[...] = jnp.dot(hh, asr[...], preferred_element_type=jnp.float32)
    ald[...] = jnp.dot(hh, adr[...], preferred_element_type=jnp.float32)
    hh8lo[...] = jnp.concatenate(
        [hh[:, 0:128], hh[:, 256:384], hh[:, 512:640], hh[:, 768:896]], axis=1)
    hh8hi[...] = jnp.concatenate(
        [hh[:, 128:256], hh[:, 384:512], hh[:, 640:768], hh[:, 896:1024]],
        axis=1)
    hhb[...] = hh.astype(jnp.bfloat16)

  bs128 = pl.BlockSpec((BN, DIN), lambda i: (i, 0))
  bs256b = pl.BlockSpec((BN, 2 * DIN), lambda i: (i, 0))
  bs512 = pl.BlockSpec((BN, 4 * DIN), lambda i: (i, 0))
  bs1024 = pl.BlockSpec((BN, 8 * DIN), lambda i: (i, 0))
  bs16 = pl.BlockSpec((BN, 16), lambda i: (i, 0))
  return pl.pallas_call(
      body,
      grid=(NB,),
      in_specs=[bs256b, bs256b, bs128, bs128, bs128,
                pl.BlockSpec((DH, DH), lambda i: (0, 0)),
                pl.BlockSpec((1, DH), lambda i: (0, 0)),
                pl.BlockSpec((DH, H * DH), lambda i: (0, 0)),
                pl.BlockSpec((H * DH, 16), lambda i: (0, 0)),
                pl.BlockSpec((H * DH, 16), lambda i: (0, 0))],
      out_specs=[bs512, bs512, bs1024, bs16, bs16],
      out_shape=[
          jax.ShapeDtypeStruct((N, 4 * DIN), jnp.float32),
          jax.ShapeDtypeStruct((N, 4 * DIN), jnp.float32),
          jax.ShapeDtypeStruct((N, 8 * DIN), jnp.bfloat16),
          jax.ShapeDtypeStruct((N, 16), jnp.float32),
          jax.ShapeDtypeStruct((N, 16), jnp.float32),
      ],
  )(ag0f, ag1f, lo, hi, dv, W2, b2, W3, As, Ad)


def _k3b(als, ald):
  def body(alsr, aldr, ms, md, cout):
    i = pl.program_id(0)

    @pl.when(i == 0)
    def _():
      ms[...] = jnp.full((1, 16), -1e30, jnp.float32)
      md[...] = jnp.full((1, 16), -1e30, jnp.float32)

    ms[...] = jnp.maximum(ms[...], jnp.max(alsr[...], axis=0, keepdims=True))
    md[...] = jnp.maximum(md[...], jnp.max(aldr[...], axis=0, keepdims=True))

    @pl.when(i == NB - 1)
    def _():
      a = ms[...]
      b = md[...]
      cout[...] = jnp.maximum(a, 0.2 * a) + jnp.maximum(b, 0.2 * b)

  bs16 = pl.BlockSpec((BN, 16), lambda i: (i, 0))
  os = pl.BlockSpec((1, 16), lambda i: (0, 0))
  return pl.pallas_call(
      body,
      grid=(NB,),
      in_specs=[bs16, bs16],
      out_specs=[os, os, os],
      out_shape=[jax.ShapeDtypeStruct((1, 16), jnp.float32)] * 3,
  )(als, ald)


def _k4(s0, s1, als, ald, C):
  def body(s0r, s1r, alsr, aldr, cr, r8, ws8):
    t = alsr[...] + aldr[...]
    t = jnp.maximum(t, 0.2 * t) - cr[...]
    exs = jnp.exp(t)
    stot = s0r[...] + s1r[...] + exs
    lane = lax.broadcasted_iota(jnp.int32, (BN, 16), 1)
    r = jnp.where(lane < H, 1.0 / (stot + 1e-16), 0.0)
    r8[...] = r
    ws8[...] = exs * r

  bs16 = pl.BlockSpec((BN, 16), lambda i: (i, 0))
  return pl.pallas_call(
      body,
      grid=(NB,),
      in_specs=[bs16, bs16, bs16, bs16,
                pl.BlockSpec((1, 16), lambda i: (0, 0))],
      out_specs=[bs16, bs16],
      out_shape=[jax.ShapeDtypeStruct((N, 16), jnp.float32)] * 2,
  )(s0, s1, als, ald, C)


def _k5(ag0, ag1, hlo, hhi, r8, ws8, b3f, batchT):
  def body(a0r, a1r, hlor, hhir, r8r, ws8r, b3r, btr, pool, cnt):
    i = pl.program_id(0)
    agg = a0r[...].astype(jnp.float32) + a1r[...].astype(jnp.float32)
    ws = ws8r[...]
    hlo_v = hlor[...]
    hhi_v = hhir[...]
    self_lo = ws[:, 0:1] * hlo_v[:, 0:128]
    self_hi = ws[:, 0:1] * hhi_v[:, 0:128]
    for h in range(1, H):
      self_lo = self_lo + ws[:, h:h + 1] * hlo_v[:, h * 128:(h + 1) * 128]
      self_hi = self_hi + ws[:, h:h + 1] * hhi_v[:, h * 128:(h + 1) * 128]
    m = 0.25 * (agg + jnp.concatenate([self_lo, self_hi], axis=1))
    h3 = _relu(m + b3r[...])
    mask = (lax.broadcasted_iota(jnp.int32, (BN, G), 1) == btr[...])
    mf = jnp.where(mask, 1.0, 0.0)

    @pl.when(i == 0)
    def _():
      pool[...] = jnp.zeros((G, DH), jnp.float32)
      cnt[...] = jnp.zeros((G, 8), jnp.float32)

    dn = (((0,), (0,)), ((), ()))
    pool[...] += lax.dot_general(mf, h3, dn,
                                 preferred_element_type=jnp.float32)
    cnt[...] += lax.dot_general(mf, jnp.ones((BN, 8), jnp.float32), dn,
                                preferred_element_type=jnp.float32)

  bs256b = pl.BlockSpec((BN, 2 * DIN), lambda i: (i, 0))
  bs512 = pl.BlockSpec((BN, 4 * DIN), lambda i: (i, 0))
  bs16 = pl.BlockSpec((BN, 16), lambda i: (i, 0))
  return pl.pallas_call(
      body,
      grid=(NB,),
      in_specs=[bs256b, bs256b, bs512, bs512, bs16, bs16,
                pl.BlockSpec((1, DH), lambda i: (0, 0)),
                pl.BlockSpec((BN, 1), lambda i: (i, 0))],
      out_specs=[pl.BlockSpec((G, DH), lambda i: (0, 0)),
                 pl.BlockSpec((G, 8), lambda i: (0, 0))],
      out_shape=[
          jax.ShapeDtypeStruct((G, DH), jnp.float32),
          jax.ShapeDtypeStruct((G, 8), jnp.float32),
      ],
  )(ag0, ag1, hlo, hhi, r8, ws8, b3f, batchT)


def _k6(pool, cnt, W4, b4):
  def body(poolr, cntr, w4, b4r, out):
    gr = poolr[...] / jnp.maximum(cntr[:, :1], 1.0)
    out[...] = jnp.dot(gr, w4[...], preferred_element_type=jnp.float32) + b4r[...]

  return pl.pallas_call(
      body,
      in_specs=[pl.BlockSpec((G, DH), lambda: (0, 0)),
                pl.BlockSpec((G, 8), lambda: (0, 0)),
                pl.BlockSpec((DH, 4), lambda: (0, 0)),
                pl.BlockSpec((1, 4), lambda: (0, 0))],
      out_specs=pl.BlockSpec((G, 4), lambda: (0, 0)),
      out_shape=jax.ShapeDtypeStruct((G, 4), jnp.float32),
  )(pool, cnt, W4, b4)


def kernel(x, edge_index, batch, W1, b1, W2, b2, W3, a_src, a_dst, b3,
           We, be, Wm, bm, Wb, bb, Wp, bp):
  npad = EP - E
  src = jnp.concatenate([edge_index[0], jnp.zeros((npad,), jnp.int32)])
  dst = jnp.concatenate([edge_index[1], jnp.full((npad,), N, jnp.int32)])

  zeros16 = jnp.zeros((NPS, 16), jnp.float32)
  zeros128 = jnp.zeros((NPS, DIN), jnp.float32)
  ones_ck = jnp.ones((CKC, 16), jnp.float32)

  As = jnp.zeros((H * DH, 16), jnp.float32)
  Ad = jnp.zeros((H * DH, 16), jnp.float32)
  for h in range(H):
    As = As.at[h * DH:(h + 1) * DH, h].set(a_src[h])
    Ad = Ad.at[h * DH:(h + 1) * DH, h].set(a_dst[h])

  W4 = jnp.concatenate([We, Wm, Wb, Wp], axis=1)
  b4 = jnp.concatenate([be, bm, bb, bp]).reshape(1, 4)

  degp = _sc_count(dst, ones_ck, zeros16)
  xp, dv = _k1(degp[:N], degp[N:], x)
  a1p = _sc_seg_rows(xp, src, dst, zeros128)
  h1plo, h1phi, h1pb = _k2(a1p[:N], a1p[N:], xp, dv, W1, b1.reshape(1, DH))

  zeros256b = jnp.zeros((NPS, 2 * DIN), jnp.bfloat16)
  a2 = _sc_seg_rows_b(h1pb, src, dst, zeros256b)
  hh8lo, hh8hi, hhb, als, ald = _k3(
      a2[:N], a2[N:], h1plo, h1phi, dv, W2, b2.reshape(1, DH), W3, As, Ad)

  _, _, C = _k3b(als, ald)
  pad8 = jnp.zeros((NA - N, 16), jnp.float32)
  ex, sp = _sc_gat_logits(jnp.concatenate([als, pad8]),
                          jnp.concatenate([ald, pad8]),
                          C.reshape(16), src, dst, zeros16)
  r8, ws8 = _k4(sp[:N], sp[N:], als, ald, C)
  r8p = jnp.concatenate([r8, pad8])
  a3 = _sc_gat_agg(hhb, ex, r8p, src, dst, zeros256b)

  pool, cnt = _k5(a3[:N], a3[N:], hh8lo, hh8hi,
                  r8, ws8, b3.reshape(1, DH), batch.reshape(N, 1))
  return _k6(pool, cnt, W4, b4)
